# Initial kernel scaffold; baseline (speedup 1.0000x reference)
import functools

import jax
import jax.numpy as jnp
from jax import lax
from jax.experimental import pallas as pl
from jax.experimental.pallas import tpu as pltpu
from jax.experimental.pallas import tpu_sc as plsc

NC = 2
NS = 16
L = 16
NW = NC * NS
EB = 128
D = 128


def _mm2_body(x_ref, wl_ref, bl_ref, wr_ref, br_ref, xl_ref, xr_ref):
    x = x_ref[...]
    xl_ref[...] = jnp.dot(x, wl_ref[...], preferred_element_type=jnp.float32) + bl_ref[...]
    xr_ref[...] = jnp.dot(x, wr_ref[...], preferred_element_type=jnp.float32) + br_ref[...]


def _merge_mm2_body(pa_ref, pb_ref, b_ref, wl_ref, bl_ref, wr_ref, br_ref,
                    xl_ref, xr_ref):
    h = jnp.maximum(pa_ref[...] + pb_ref[...] + b_ref[...], 0.0)
    xl_ref[...] = jnp.dot(h, wl_ref[...], preferred_element_type=jnp.float32) + bl_ref[...]
    xr_ref[...] = jnp.dot(h, wr_ref[...], preferred_element_type=jnp.float32) + br_ref[...]


def _make_mm2(n, rb):
    grid = n // rb
    full = lambda i: (0, 0)
    row = lambda i: (i, 0)
    return pl.pallas_call(
        _mm2_body,
        grid=(grid,),
        in_specs=[
            pl.BlockSpec((rb, D), row),
            pl.BlockSpec((D, D), full),
            pl.BlockSpec((1, D), full),
            pl.BlockSpec((D, D), full),
            pl.BlockSpec((1, D), full),
        ],
        out_specs=[pl.BlockSpec((rb, D), row), pl.BlockSpec((rb, D), row)],
        out_shape=[
            jax.ShapeDtypeStruct((n, D), jnp.float32),
            jax.ShapeDtypeStruct((n, D), jnp.float32),
        ],
    )


def _make_merge_mm2(n, rb):
    grid = n // rb
    full = lambda i: (0, 0)
    row = lambda i: (i, 0)
    return pl.pallas_call(
        _merge_mm2_body,
        grid=(grid,),
        in_specs=[
            pl.BlockSpec((rb, D), row),
            pl.BlockSpec((rb, D), row),
            pl.BlockSpec((1, D), full),
            pl.BlockSpec((D, D), full),
            pl.BlockSpec((1, D), full),
            pl.BlockSpec((D, D), full),
            pl.BlockSpec((1, D), full),
        ],
        out_specs=[pl.BlockSpec((rb, D), row), pl.BlockSpec((rb, D), row)],
        out_shape=[
            jax.ShapeDtypeStruct((n, D), jnp.float32),
            jax.ShapeDtypeStruct((n, D), jnp.float32),
        ],
    )


def _make_merge_pool(n, rb, ng):
    grid = n // rb

    def body(pa_ref, pb_ref, b_ref, bt_ref, pooled_ref, h_ref, acc, cnt):
        i = pl.program_id(0)
        h = pa_ref[...] + pb_ref[...] + b_ref[...]
        h_ref[...] = h
        groups = lax.broadcasted_iota(jnp.int32, (rb, ng), 1)
        oh = (bt_ref[...] == groups).astype(jnp.float32)
        dn = (((0,), (0,)), ((), ()))
        ps = lax.dot_general(oh, h, dn, preferred_element_type=jnp.float32)
        cs = lax.dot_general(oh, jnp.ones((rb, D), jnp.float32), dn,
                             preferred_element_type=jnp.float32)

        @pl.when(i == 0)
        def _():
            acc[...] = ps
            cnt[...] = cs

        @pl.when(i > 0)
        def _():
            acc[...] = acc[...] + ps
            cnt[...] = cnt[...] + cs

        @pl.when(i == grid - 1)
        def _():
            pooled_ref[...] = acc[...] / jnp.maximum(cnt[...], 1.0)

    full = lambda i: (0, 0)
    row = lambda i: (i, 0)
    return pl.pallas_call(
        body,
        grid=(grid,),
        in_specs=[
            pl.BlockSpec((rb, D), row),
            pl.BlockSpec((rb, D), row),
            pl.BlockSpec((1, D), full),
            pl.BlockSpec((rb, 1), row),
        ],
        out_specs=[pl.BlockSpec((ng, D), full), pl.BlockSpec((rb, D), row)],
        out_shape=[
            jax.ShapeDtypeStruct((ng, D), jnp.float32),
            jax.ShapeDtypeStruct((n, D), jnp.float32),
        ],
        scratch_shapes=[
            pltpu.VMEM((ng, D), jnp.float32),
            pltpu.VMEM((ng, D), jnp.float32),
        ],
    )


def _leaky(t):
    return jnp.maximum(t, 0.0) + 0.2 * jnp.minimum(t, 0.0)


@functools.lru_cache(maxsize=None)
def _make_sc_pass1(etot, epad, np_):
    nblk = epad // (NW * EB)
    epw = nblk * EB
    slc = np_ // NS
    mesh = plsc.VectorSubcoreMesh(core_axis_name="c", subcore_axis_name="s",
                                  num_cores=NC, num_subcores=NS)

    @functools.partial(
        pl.kernel,
        out_type=[
            jax.ShapeDtypeStruct((epad,), jnp.float32),
            jax.ShapeDtypeStruct((NC, np_), jnp.float32),
        ],
        mesh=mesh,
        scratch_types=[
            pltpu.VMEM((D,), jnp.float32),
            pltpu.VMEM((EB,), jnp.int32),
            pltpu.VMEM((EB,), jnp.int32),
            pltpu.VMEM((EB, D), jnp.float32),
            pltpu.VMEM((EB, D), jnp.float32),
            pltpu.VMEM((EB,), jnp.float32),
            pltpu.VMEM((640,), jnp.float32),
            pltpu.VMEM_SHARED((10240,), jnp.float32),
            pltpu.SemaphoreType.DMA,
            pltpu.SemaphoreType.DMA,
        ],
    )
    def pass1(xl_hbm, xr_hbm, s_hbm, d_hbm, att_hbm, ee_hbm, dparts_hbm,
              att_v, sidx, didx, rows_s, rows_d, eev, zb, dsh, sem_s, sem_d):
        c = lax.axis_index("c")
        tid = lax.axis_index("s")
        wid = tid * NC + c

        def zloop(i, _):
            zb[pl.ds(i * L, L)] = jnp.zeros((L,), jnp.float32)
            return 0
        lax.fori_loop(0, slc // L, zloop, 0)
        pltpu.sync_copy(zb, dsh.at[pl.ds(tid * slc, slc)])
        pltpu.sync_copy(att_hbm, att_v)
        plsc.subcore_barrier()

        base = wid * epw
        lanes = lax.iota(jnp.int32, (L,))

        def blk(b, _):
            off = pl.multiple_of(base + b * EB, EB)
            pltpu.sync_copy(s_hbm.at[pl.ds(off, EB)], sidx)
            pltpu.sync_copy(d_hbm.at[pl.ds(off, EB)], didx)
            cps = pltpu.async_copy(xl_hbm.at[sidx], rows_s, sem_s)
            cpd = pltpu.async_copy(xr_hbm.at[didx], rows_d, sem_d)
            cps.wait()
            cpd.wait()

            def sub(t, _):
                jvec = t * L + lanes
                acc = jnp.zeros((L,), jnp.float32)
                for k in range(D):
                    kvec = jnp.full((L,), k, jnp.int32)
                    vs = plsc.load_gather(rows_s, [jvec, kvec])
                    vd = plsc.load_gather(rows_d, [jvec, kvec])
                    acc = acc + att_v[k] * _leaky(vs + vd)
                gid = off + jvec
                val = jnp.where(gid < etot, jnp.exp(acc), 0.0)
                eev[pl.ds(t * L, L)] = val
                return 0
            lax.fori_loop(0, EB // L, sub, 0)

            pltpu.sync_copy(eev, ee_hbm.at[pl.ds(off, EB)])
            pltpu.sync_copy(eev, dsh.at[didx], add=True)
            return 0
        lax.fori_loop(0, nblk, blk, 0)

        plsc.subcore_barrier()
        pltpu.sync_copy(dsh.at[pl.ds(tid * slc, slc)],
                        dparts_hbm.at[c, pl.ds(tid * slc, slc)])

    return pass1


@functools.lru_cache(maxsize=None)
def _make_sc_pass2(etot, epad, np_):
    nblk = epad // (NW * EB)
    epw = nblk * EB
    slc = np_ // NS
    zr = 128
    mesh = plsc.VectorSubcoreMesh(core_axis_name="c", subcore_axis_name="s",
                                  num_cores=NC, num_subcores=NS)

    @functools.partial(
        pl.kernel,
        out_type=jax.ShapeDtypeStruct((NC, np_, D), jnp.float32),
        mesh=mesh,
        scratch_types=[
            pltpu.VMEM((EB,), jnp.int32),
            pltpu.VMEM((EB,), jnp.int32),
            pltpu.VMEM((EB, D), jnp.float32),
            pltpu.VMEM((EB,), jnp.float32),
            pltpu.VMEM((EB,), jnp.float32),
            pltpu.VMEM((10240,), jnp.float32),
            pltpu.VMEM((10240,), jnp.float32),
            pltpu.VMEM((128, D), jnp.float32),
            pltpu.VMEM_SHARED((10240, D), jnp.float32),
            pltpu.SemaphoreType.DMA,
        ],
    )
    def pass2(xl_hbm, ee_hbm, s_hbm, d_hbm, dparts_hbm, out_hbm,
              sidx, didx, rows, eev, alv, dn0, dn1, zb, osh, sem):
        c = lax.axis_index("c")
        tid = lax.axis_index("s")
        wid = tid * NC + c

        pltpu.sync_copy(dparts_hbm.at[0], dn0)
        pltpu.sync_copy(dparts_hbm.at[1], dn1)

        def dmerge(i, _):
            sl = pl.ds(i * L, L)
            dn0[sl] = dn0[sl] + dn1[sl]
            return 0
        lax.fori_loop(0, np_ // L, dmerge, 0)

        zr = 128

        def zrow(i, _):
            for q in range(D // L):
                zb[i, pl.ds(q * L, L)] = jnp.zeros((L,), jnp.float32)
            return 0
        lax.fori_loop(0, zr, zrow, 0)

        def zcp(i, _):
            pltpu.sync_copy(zb, osh.at[pl.ds(tid * slc + i * zr, zr), :])
            return 0
        lax.fori_loop(0, slc // zr, zcp, 0)
        plsc.subcore_barrier()

        base = wid * epw
        lanes = lax.iota(jnp.int32, (L,))

        def blk(b, _):
            off = pl.multiple_of(base + b * EB, EB)
            pltpu.sync_copy(s_hbm.at[pl.ds(off, EB)], sidx)
            pltpu.sync_copy(d_hbm.at[pl.ds(off, EB)], didx)
            pltpu.sync_copy(ee_hbm.at[pl.ds(off, EB)], eev)
            pltpu.async_copy(xl_hbm.at[sidx], rows, sem).wait()

            def sub(t, _):
                sl = pl.ds(t * L, L)
                dv = plsc.load_gather(dn0, [didx[sl]])
                alv[sl] = eev[sl] / (dv + 1e-16)
                return 0
            lax.fori_loop(0, EB // L, sub, 0)

            def wrow(j, _):
                a = alv[j]
                for q in range(D // L):
                    sl = pl.ds(q * L, L)
                    rows[j, sl] = rows[j, sl] * a
                return 0
            lax.fori_loop(0, EB, wrow, 0)

            pltpu.sync_copy(rows, osh.at[didx], add=True)
            return 0
        lax.fori_loop(0, nblk, blk, 0)

        plsc.subcore_barrier()
        pltpu.sync_copy(osh.at[pl.ds(tid * slc, slc), :],
                        out_hbm.at[c, pl.ds(tid * slc, slc), :])

    return pass2


def kernel(x, edge_index, batch, Wl1, bl1, Wr1, br1, att1, b1,
           Wl2, bl2, Wr2, br2, att2, b2):
    n = x.shape[0]
    e = edge_index.shape[1]
    etot = e + n
    epad = ((etot + NW * EB - 1) // (NW * EB)) * (NW * EB)
    np_ = ((n + NS * EB - 1) // (NS * EB)) * (NS * EB)
    ng = 16

    loop = jnp.arange(n, dtype=jnp.int32)
    zpad = jnp.zeros((epad - etot,), jnp.int32)
    s = jnp.concatenate([edge_index[0], loop, zpad])
    d = jnp.concatenate([edge_index[1], loop, zpad])

    pass1 = _make_sc_pass1(etot, epad, np_)
    pass2 = _make_sc_pass2(etot, epad, np_)
    mm2 = _make_mm2(n, 1000)
    merge_mm2 = _make_merge_mm2(n, 1000)
    merge_pool = _make_merge_pool(n, 1000, ng)

    b1r = b1.reshape(1, D)
    b2r = b2.reshape(1, D)

    xl1, xr1 = mm2(x, Wl1, bl1.reshape(1, D), Wr1, br1.reshape(1, D))
    ee1, dparts1 = pass1(xl1, xr1, s, d, att1.reshape(D))
    parts1 = pass2(xl1, ee1, s, d, dparts1)

    xl2, xr2 = merge_mm2(parts1[0, :n], parts1[1, :n], b1r,
                         Wl2, bl2.reshape(1, D), Wr2, br2.reshape(1, D))
    ee2, dparts2 = pass1(xl2, xr2, s, d, att2.reshape(D))
    parts2 = pass2(xl2, ee2, s, d, dparts2)

    pooled, h = merge_pool(parts2[0, :n], parts2[1, :n], b2r,
                           batch.reshape(n, 1))
    return (pooled, h)


# speedup vs baseline: 8.5843x; 8.5843x over previous
import functools

import jax
import jax.numpy as jnp
from jax import lax
from jax.experimental import pallas as pl
from jax.experimental.pallas import tpu as pltpu
from jax.experimental.pallas import tpu_sc as plsc

NC = 2
NS = 16
L = 16
NW = NC * NS
EB = 128
D = 128


def _mm2_body(x_ref, wl_ref, bl_ref, wr_ref, br_ref, xl_ref, xr_ref):
    x = x_ref[...]
    xl_ref[...] = jnp.dot(x, wl_ref[...], preferred_element_type=jnp.float32) + bl_ref[...]
    xr_ref[...] = jnp.dot(x, wr_ref[...], preferred_element_type=jnp.float32) + br_ref[...]


def _merge_mm2_body(pa_ref, pb_ref, b_ref, wl_ref, bl_ref, wr_ref, br_ref,
                    xl_ref, xr_ref):
    h = jnp.maximum(pa_ref[...] + pb_ref[...] + b_ref[...], 0.0)
    xl_ref[...] = jnp.dot(h, wl_ref[...], preferred_element_type=jnp.float32) + bl_ref[...]
    xr_ref[...] = jnp.dot(h, wr_ref[...], preferred_element_type=jnp.float32) + br_ref[...]


def _make_mm2(n, rb):
    grid = n // rb
    full = lambda i: (0, 0)
    row = lambda i: (i, 0)
    return pl.pallas_call(
        _mm2_body,
        grid=(grid,),
        in_specs=[
            pl.BlockSpec((rb, D), row),
            pl.BlockSpec((D, D), full),
            pl.BlockSpec((1, D), full),
            pl.BlockSpec((D, D), full),
            pl.BlockSpec((1, D), full),
        ],
        out_specs=[pl.BlockSpec((rb, D), row), pl.BlockSpec((rb, D), row)],
        out_shape=[
            jax.ShapeDtypeStruct((n, D), jnp.float32),
            jax.ShapeDtypeStruct((n, D), jnp.float32),
        ],
    )


def _make_merge_mm2(n, rb):
    grid = n // rb
    full = lambda i: (0, 0)
    row = lambda i: (i, 0)
    return pl.pallas_call(
        _merge_mm2_body,
        grid=(grid,),
        in_specs=[
            pl.BlockSpec((rb, D), row),
            pl.BlockSpec((rb, D), row),
            pl.BlockSpec((1, D), full),
            pl.BlockSpec((D, D), full),
            pl.BlockSpec((1, D), full),
            pl.BlockSpec((D, D), full),
            pl.BlockSpec((1, D), full),
        ],
        out_specs=[pl.BlockSpec((rb, D), row), pl.BlockSpec((rb, D), row)],
        out_shape=[
            jax.ShapeDtypeStruct((n, D), jnp.float32),
            jax.ShapeDtypeStruct((n, D), jnp.float32),
        ],
    )


def _make_merge_pool(n, rb, ng):
    grid = n // rb

    def body(pa_ref, pb_ref, b_ref, bt_ref, pooled_ref, h_ref, acc, cnt):
        i = pl.program_id(0)
        h = pa_ref[...] + pb_ref[...] + b_ref[...]
        h_ref[...] = h
        groups = lax.broadcasted_iota(jnp.int32, (rb, ng), 1)
        oh = (bt_ref[...] == groups).astype(jnp.float32)
        dn = (((0,), (0,)), ((), ()))
        ps = lax.dot_general(oh, h, dn, preferred_element_type=jnp.float32)
        cs = lax.dot_general(oh, jnp.ones((rb, D), jnp.float32), dn,
                             preferred_element_type=jnp.float32)

        @pl.when(i == 0)
        def _():
            acc[...] = ps
            cnt[...] = cs

        @pl.when(i > 0)
        def _():
            acc[...] = acc[...] + ps
            cnt[...] = cnt[...] + cs

        @pl.when(i == grid - 1)
        def _():
            pooled_ref[...] = acc[...] / jnp.maximum(cnt[...], 1.0)

    full = lambda i: (0, 0)
    row = lambda i: (i, 0)
    return pl.pallas_call(
        body,
        grid=(grid,),
        in_specs=[
            pl.BlockSpec((rb, D), row),
            pl.BlockSpec((rb, D), row),
            pl.BlockSpec((1, D), full),
            pl.BlockSpec((rb, 1), row),
        ],
        out_specs=[pl.BlockSpec((ng, D), full), pl.BlockSpec((rb, D), row)],
        out_shape=[
            jax.ShapeDtypeStruct((ng, D), jnp.float32),
            jax.ShapeDtypeStruct((n, D), jnp.float32),
        ],
        scratch_shapes=[
            pltpu.VMEM((ng, D), jnp.float32),
            pltpu.VMEM((ng, D), jnp.float32),
        ],
    )


def _leaky(t):
    return jnp.maximum(t, 0.0) + 0.2 * jnp.minimum(t, 0.0)


@functools.lru_cache(maxsize=None)
def _make_sc_pass1(etot, epad, np_):
    nblk = epad // (NW * EB)
    epw = nblk * EB
    slc = np_ // NS
    mesh = plsc.VectorSubcoreMesh(core_axis_name="c", subcore_axis_name="s",
                                  num_cores=NC, num_subcores=NS)

    @functools.partial(
        pl.kernel,
        out_type=[
            jax.ShapeDtypeStruct((epad,), jnp.float32),
            jax.ShapeDtypeStruct((NC, np_), jnp.float32),
        ],
        mesh=mesh,
        compiler_params=pltpu.CompilerParams(needs_layout_passes=False),
        scratch_types=[
            pltpu.VMEM((D,), jnp.float32),
            pltpu.VMEM((EB,), jnp.int32),
            pltpu.VMEM((EB,), jnp.int32),
            pltpu.VMEM((EB, D), jnp.float32),
            pltpu.VMEM((EB, D), jnp.float32),
            pltpu.VMEM((EB,), jnp.float32),
            pltpu.VMEM((slc,), jnp.float32),
            pltpu.VMEM_SHARED((np_,), jnp.float32),
            pltpu.SemaphoreType.DMA,
            pltpu.SemaphoreType.DMA,
        ],
    )
    def pass1(xl_hbm, xr_hbm, s_hbm, d_hbm, att_hbm, ee_hbm, dparts_hbm,
              att_v, sidx, didx, rows_s, rows_d, eev, zb, dsh, sem_s, sem_d):
        c = lax.axis_index("c")
        tid = lax.axis_index("s")
        wid = tid * NC + c

        def zloop(i, _):
            zb[pl.ds(i * L, L)] = jnp.zeros((L,), jnp.float32)
            return 0
        lax.fori_loop(0, slc // L, zloop, 0)
        pltpu.sync_copy(zb, dsh.at[pl.ds(tid * slc, slc)])
        pltpu.sync_copy(att_hbm, att_v)
        plsc.subcore_barrier()

        base = wid * epw
        lanes = lax.iota(jnp.int32, L)
        attv = [att_v[pl.ds(q * L, L)] for q in range(D // L)]

        def blk(b, _):
            off = pl.multiple_of(base + b * EB, EB)
            pltpu.sync_copy(s_hbm.at[pl.ds(off, EB)], sidx)
            pltpu.sync_copy(d_hbm.at[pl.ds(off, EB)], didx)
            cps = pltpu.async_copy(xl_hbm.at[sidx], rows_s, sem_s)
            cpd = pltpu.async_copy(xr_hbm.at[didx], rows_d, sem_d)
            cps.wait()
            cpd.wait()

            def sub(t, _):
                evec = jnp.zeros((L,), jnp.float32)
                for jj in range(L):
                    j = t * L + jj
                    acc = jnp.zeros((L,), jnp.float32)
                    for q in range(D // L):
                        sl = pl.ds(q * L, L)
                        acc = acc + attv[q] * _leaky(rows_s[j, sl] + rows_d[j, sl])
                    evec = jnp.where(lanes == jj, jnp.sum(acc), evec)
                gid = off + t * L + lanes
                val = jnp.where(gid < etot, jnp.exp(evec), 0.0)
                eev[pl.ds(t * L, L)] = val
                return 0
            lax.fori_loop(0, EB // L, sub, 0)

            pltpu.sync_copy(eev, ee_hbm.at[pl.ds(off, EB)])
            pltpu.sync_copy(eev, dsh.at[didx], add=True)
            return 0
        lax.fori_loop(0, nblk, blk, 0)

        plsc.subcore_barrier()
        pltpu.sync_copy(dsh.at[pl.ds(tid * slc, slc)],
                        dparts_hbm.at[c, pl.ds(tid * slc, slc)])

    return pass1


@functools.lru_cache(maxsize=None)
def _make_sc_pass2(etot, epad, np_):
    nblk = epad // (NW * EB)
    epw = nblk * EB
    slc = np_ // NS
    zr = 64
    dchunk = 1024
    mesh = plsc.VectorSubcoreMesh(core_axis_name="c", subcore_axis_name="s",
                                  num_cores=NC, num_subcores=NS)

    @functools.partial(
        pl.kernel,
        out_type=jax.ShapeDtypeStruct((NC, np_, D), jnp.float32),
        mesh=mesh,
        compiler_params=pltpu.CompilerParams(needs_layout_passes=False),
        scratch_types=[
            pltpu.VMEM((EB,), jnp.int32),
            pltpu.VMEM((EB,), jnp.int32),
            pltpu.VMEM((EB, D), jnp.float32),
            pltpu.VMEM((EB,), jnp.float32),
            pltpu.VMEM((EB,), jnp.float32),
            pltpu.VMEM((np_,), jnp.float32),
            pltpu.VMEM((dchunk,), jnp.float32),
            pltpu.VMEM((zr, D), jnp.float32),
            pltpu.VMEM_SHARED((np_, D), jnp.float32),
            pltpu.SemaphoreType.DMA,
        ],
    )
    def pass2(xl_hbm, ee_hbm, s_hbm, d_hbm, dparts_hbm, out_hbm,
              sidx, didx, rows, eev, alv, dn0, dn1, zb, osh, sem):
        c = lax.axis_index("c")
        tid = lax.axis_index("s")
        wid = tid * NC + c

        pltpu.sync_copy(dparts_hbm.at[0], dn0)

        def dmerge(ci, _):
            cbase = pl.multiple_of(ci * dchunk, dchunk)
            pltpu.sync_copy(dparts_hbm.at[1, pl.ds(cbase, dchunk)], dn1)

            def dadd(i, _):
                sl = pl.ds(cbase + i * L, L)
                dn0[sl] = dn0[sl] + dn1[pl.ds(i * L, L)]
                return 0
            lax.fori_loop(0, dchunk // L, dadd, 0)
            return 0
        lax.fori_loop(0, np_ // dchunk, dmerge, 0)

        def zrow(i, _):
            for q in range(D // L):
                zb[i, pl.ds(q * L, L)] = jnp.zeros((L,), jnp.float32)
            return 0
        lax.fori_loop(0, zr, zrow, 0)

        def zcp(i, _):
            pltpu.sync_copy(zb, osh.at[pl.ds(tid * slc + i * zr, zr), :])
            return 0
        lax.fori_loop(0, slc // zr, zcp, 0)
        plsc.subcore_barrier()

        base = wid * epw
        lanes = lax.iota(jnp.int32, L)

        def blk(b, _):
            off = pl.multiple_of(base + b * EB, EB)
            pltpu.sync_copy(s_hbm.at[pl.ds(off, EB)], sidx)
            pltpu.sync_copy(d_hbm.at[pl.ds(off, EB)], didx)
            pltpu.sync_copy(ee_hbm.at[pl.ds(off, EB)], eev)
            pltpu.async_copy(xl_hbm.at[sidx], rows, sem).wait()

            def sub(t, _):
                sl = pl.ds(t * L, L)
                dv = plsc.load_gather(dn0, [didx[sl]])
                alv[sl] = eev[sl] / (dv + 1e-16)
                return 0
            lax.fori_loop(0, EB // L, sub, 0)

            def wrow(t, _):
                av = alv[pl.ds(t * L, L)]
                for jj in range(L):
                    a = av[jj]
                    j = t * L + jj
                    for q in range(D // L):
                        sl = pl.ds(q * L, L)
                        rows[j, sl] = rows[j, sl] * a
                return 0
            lax.fori_loop(0, EB // L, wrow, 0)

            pltpu.sync_copy(rows, osh.at[didx], add=True)
            return 0
        lax.fori_loop(0, nblk, blk, 0)

        plsc.subcore_barrier()
        pltpu.sync_copy(osh.at[pl.ds(tid * slc, slc), :],
                        out_hbm.at[c, pl.ds(tid * slc, slc), :])

    return pass2


def kernel(x, edge_index, batch, Wl1, bl1, Wr1, br1, att1, b1,
           Wl2, bl2, Wr2, br2, att2, b2):
    n = x.shape[0]
    e = edge_index.shape[1]
    etot = e + n
    epad = ((etot + NW * EB - 1) // (NW * EB)) * (NW * EB)
    np_ = ((n + NS * EB - 1) // (NS * EB)) * (NS * EB)
    ng = 16

    loop = jnp.arange(n, dtype=jnp.int32)
    zpad = jnp.zeros((epad - etot,), jnp.int32)
    s = jnp.concatenate([edge_index[0], loop, zpad])
    d = jnp.concatenate([edge_index[1], loop, zpad])

    pass1 = _make_sc_pass1(etot, epad, np_)
    pass2 = _make_sc_pass2(etot, epad, np_)
    mm2 = _make_mm2(n, 1000)
    merge_mm2 = _make_merge_mm2(n, 1000)
    merge_pool = _make_merge_pool(n, 1000, ng)

    b1r = b1.reshape(1, D)
    b2r = b2.reshape(1, D)

    xl1, xr1 = mm2(x, Wl1, bl1.reshape(1, D), Wr1, br1.reshape(1, D))
    ee1, dparts1 = pass1(xl1, xr1, s, d, att1.reshape(D))
    parts1 = pass2(xl1, ee1, s, d, dparts1)

    xl2, xr2 = merge_mm2(parts1[0, :n], parts1[1, :n], b1r,
                         Wl2, bl2.reshape(1, D), Wr2, br2.reshape(1, D))
    ee2, dparts2 = pass1(xl2, xr2, s, d, att2.reshape(D))
    parts2 = pass2(xl2, ee2, s, d, dparts2)

    pooled, h = merge_pool(parts2[0, :n], parts2[1, :n], b2r,
                           batch.reshape(n, 1))
    return (pooled, h)


# speedup vs baseline: 12.9062x; 1.5035x over previous
import functools

import jax
import jax.numpy as jnp
from jax import lax
from jax.experimental import pallas as pl
from jax.experimental.pallas import tpu as pltpu
from jax.experimental.pallas import tpu_sc as plsc

NC = 2
NS = 16
L = 16
NW = NC * NS
EB = 128
D = 128


def _mm2_body(x_ref, wl_ref, bl_ref, wr_ref, br_ref, xl_ref, xr_ref):
    x = x_ref[...]
    xl_ref[...] = jnp.dot(x, wl_ref[...], preferred_element_type=jnp.float32) + bl_ref[...]
    xr_ref[...] = jnp.dot(x, wr_ref[...], preferred_element_type=jnp.float32) + br_ref[...]


def _merge_mm2_body(pa_ref, pb_ref, da_ref, db_ref, b_ref, wl_ref, bl_ref,
                    wr_ref, br_ref, xl_ref, xr_ref):
    dn = da_ref[...] + db_ref[...] + 1e-16
    h = jnp.maximum((pa_ref[...] + pb_ref[...]) / dn + b_ref[...], 0.0)
    xl_ref[...] = jnp.dot(h, wl_ref[...], preferred_element_type=jnp.float32) + bl_ref[...]
    xr_ref[...] = jnp.dot(h, wr_ref[...], preferred_element_type=jnp.float32) + br_ref[...]


def _make_mm2(n, rb):
    grid = n // rb
    full = lambda i: (0, 0)
    row = lambda i: (i, 0)
    return pl.pallas_call(
        _mm2_body,
        grid=(grid,),
        in_specs=[
            pl.BlockSpec((rb, D), row),
            pl.BlockSpec((D, D), full),
            pl.BlockSpec((1, D), full),
            pl.BlockSpec((D, D), full),
            pl.BlockSpec((1, D), full),
        ],
        out_specs=[pl.BlockSpec((rb, D), row), pl.BlockSpec((rb, D), row)],
        out_shape=[
            jax.ShapeDtypeStruct((n, D), jnp.float32),
            jax.ShapeDtypeStruct((n, D), jnp.float32),
        ],
    )


def _make_merge_mm2(n, rb):
    grid = n // rb
    full = lambda i: (0, 0)
    row = lambda i: (i, 0)
    return pl.pallas_call(
        _merge_mm2_body,
        grid=(grid,),
        in_specs=[
            pl.BlockSpec((rb, D), row),
            pl.BlockSpec((rb, D), row),
            pl.BlockSpec((rb, 1), row),
            pl.BlockSpec((rb, 1), row),
            pl.BlockSpec((1, D), full),
            pl.BlockSpec((D, D), full),
            pl.BlockSpec((1, D), full),
            pl.BlockSpec((D, D), full),
            pl.BlockSpec((1, D), full),
        ],
        out_specs=[pl.BlockSpec((rb, D), row), pl.BlockSpec((rb, D), row)],
        out_shape=[
            jax.ShapeDtypeStruct((n, D), jnp.float32),
            jax.ShapeDtypeStruct((n, D), jnp.float32),
        ],
    )


def _make_merge_pool(n, rb, ng):
    grid = n // rb

    def body(pa_ref, pb_ref, da_ref, db_ref, b_ref, bt_ref, pooled_ref, h_ref,
             acc, cnt):
        i = pl.program_id(0)
        dn = da_ref[...] + db_ref[...] + 1e-16
        h = (pa_ref[...] + pb_ref[...]) / dn + b_ref[...]
        h_ref[...] = h
        groups = lax.broadcasted_iota(jnp.int32, (rb, ng), 1)
        oh = (bt_ref[...] == groups).astype(jnp.float32)
        dn = (((0,), (0,)), ((), ()))
        ps = lax.dot_general(oh, h, dn, preferred_element_type=jnp.float32)
        cs = lax.dot_general(oh, jnp.ones((rb, D), jnp.float32), dn,
                             preferred_element_type=jnp.float32)

        @pl.when(i == 0)
        def _():
            acc[...] = ps
            cnt[...] = cs

        @pl.when(i > 0)
        def _():
            acc[...] = acc[...] + ps
            cnt[...] = cnt[...] + cs

        @pl.when(i == grid - 1)
        def _():
            pooled_ref[...] = acc[...] / jnp.maximum(cnt[...], 1.0)

    full = lambda i: (0, 0)
    row = lambda i: (i, 0)
    return pl.pallas_call(
        body,
        grid=(grid,),
        in_specs=[
            pl.BlockSpec((rb, D), row),
            pl.BlockSpec((rb, D), row),
            pl.BlockSpec((rb, 1), row),
            pl.BlockSpec((rb, 1), row),
            pl.BlockSpec((1, D), full),
            pl.BlockSpec((rb, 1), row),
        ],
        out_specs=[pl.BlockSpec((ng, D), full), pl.BlockSpec((rb, D), row)],
        out_shape=[
            jax.ShapeDtypeStruct((ng, D), jnp.float32),
            jax.ShapeDtypeStruct((n, D), jnp.float32),
        ],
        scratch_shapes=[
            pltpu.VMEM((ng, D), jnp.float32),
            pltpu.VMEM((ng, D), jnp.float32),
        ],
    )


def _leaky(t):
    return jnp.maximum(t, 0.0) + 0.2 * jnp.minimum(t, 0.0)


@functools.lru_cache(maxsize=None)
def _make_sc_layer(etot, epad, np_):
    nblk = epad // (NW * EB)
    epw = nblk * EB
    slc = np_ // NS
    zr = 64
    mesh = plsc.VectorSubcoreMesh(core_axis_name="c", subcore_axis_name="s",
                                  num_cores=NC, num_subcores=NS)

    @functools.partial(
        pl.kernel,
        out_type=[
            jax.ShapeDtypeStruct((NC, np_, D), jnp.float32),
            jax.ShapeDtypeStruct((NC, np_), jnp.float32),
        ],
        mesh=mesh,
        compiler_params=pltpu.CompilerParams(needs_layout_passes=False),
        scratch_types=[
            pltpu.VMEM((D,), jnp.float32),
            pltpu.VMEM((EB,), jnp.int32),
            pltpu.VMEM((EB,), jnp.int32),
            pltpu.VMEM((EB, D), jnp.float32),
            pltpu.VMEM((EB, D), jnp.float32),
            pltpu.VMEM((EB,), jnp.float32),
            pltpu.VMEM((zr, D), jnp.float32),
            pltpu.VMEM((slc,), jnp.float32),
            pltpu.VMEM_SHARED((np_, D), jnp.float32),
            pltpu.VMEM_SHARED((np_,), jnp.float32),
            pltpu.SemaphoreType.DMA,
            pltpu.SemaphoreType.DMA,
        ],
    )
    def fused(xl_hbm, xr_hbm, s_hbm, d_hbm, att_hbm, out_hbm, dparts_hbm,
              att_v, sidx, didx, rows_s, rows_d, eev, zb, zb1, osh, dsh,
              sem_s, sem_d):
        c = lax.axis_index("c")
        tid = lax.axis_index("s")
        wid = tid * NC + c

        def zrow(i, _):
            for q in range(D // L):
                zb[i, pl.ds(q * L, L)] = jnp.zeros((L,), jnp.float32)
            return 0
        lax.fori_loop(0, zr, zrow, 0)

        def z1(i, _):
            zb1[pl.ds(i * L, L)] = jnp.zeros((L,), jnp.float32)
            return 0
        lax.fori_loop(0, slc // L, z1, 0)

        def zcp(i, _):
            pltpu.sync_copy(zb, osh.at[pl.ds(tid * slc + i * zr, zr), :])
            return 0
        lax.fori_loop(0, slc // zr, zcp, 0)
        pltpu.sync_copy(zb1, dsh.at[pl.ds(tid * slc, slc)])
        pltpu.sync_copy(att_hbm, att_v)
        plsc.subcore_barrier()

        base = wid * epw
        lanes = lax.iota(jnp.int32, L)
        attv = [att_v[pl.ds(q * L, L)] for q in range(D // L)]

        def blk(b, _):
            off = pl.multiple_of(base + b * EB, EB)
            pltpu.sync_copy(s_hbm.at[pl.ds(off, EB)], sidx)
            pltpu.sync_copy(d_hbm.at[pl.ds(off, EB)], didx)
            cps = pltpu.async_copy(xl_hbm.at[sidx], rows_s, sem_s)
            cpd = pltpu.async_copy(xr_hbm.at[didx], rows_d, sem_d)
            cps.wait()
            cpd.wait()

            def sub(t, _):
                evec = jnp.zeros((L,), jnp.float32)
                for jj in range(L):
                    j = t * L + jj
                    acc = jnp.zeros((L,), jnp.float32)
                    for q in range(D // L):
                        sl = pl.ds(q * L, L)
                        acc = acc + attv[q] * _leaky(rows_s[j, sl] + rows_d[j, sl])
                    evec = jnp.where(lanes == jj, jnp.sum(acc), evec)
                gid = off + t * L + lanes
                ee = jnp.where(gid < etot, jnp.exp(evec), 0.0)
                eev[pl.ds(t * L, L)] = ee
                for jj in range(L):
                    a = ee[jj]
                    j = t * L + jj
                    for q in range(D // L):
                        sl = pl.ds(q * L, L)
                        rows_s[j, sl] = rows_s[j, sl] * a
                return 0
            lax.fori_loop(0, EB // L, sub, 0)

            pltpu.sync_copy(eev, dsh.at[didx], add=True)
            pltpu.sync_copy(rows_s, osh.at[didx], add=True)
            return 0
        lax.fori_loop(0, nblk, blk, 0)

        plsc.subcore_barrier()
        pltpu.sync_copy(osh.at[pl.ds(tid * slc, slc), :],
                        out_hbm.at[c, pl.ds(tid * slc, slc), :])
        pltpu.sync_copy(dsh.at[pl.ds(tid * slc, slc)],
                        dparts_hbm.at[c, pl.ds(tid * slc, slc)])

    return fused


def kernel(x, edge_index, batch, Wl1, bl1, Wr1, br1, att1, b1,
           Wl2, bl2, Wr2, br2, att2, b2):
    n = x.shape[0]
    e = edge_index.shape[1]
    etot = e + n
    epad = ((etot + NW * EB - 1) // (NW * EB)) * (NW * EB)
    np_ = ((n + NS * EB - 1) // (NS * EB)) * (NS * EB)
    ng = 16

    loop = jnp.arange(n, dtype=jnp.int32)
    zpad = jnp.zeros((epad - etot,), jnp.int32)
    s = jnp.concatenate([edge_index[0], loop, zpad])
    d = jnp.concatenate([edge_index[1], loop, zpad])

    sc_layer = _make_sc_layer(etot, epad, np_)
    mm2 = _make_mm2(n, 1000)
    merge_mm2 = _make_merge_mm2(n, 1000)
    merge_pool = _make_merge_pool(n, 1000, ng)

    b1r = b1.reshape(1, D)
    b2r = b2.reshape(1, D)

    xl1, xr1 = mm2(x, Wl1, bl1.reshape(1, D), Wr1, br1.reshape(1, D))
    oparts1, dparts1 = sc_layer(xl1, xr1, s, d, att1.reshape(D))

    xl2, xr2 = merge_mm2(oparts1[0, :n], oparts1[1, :n],
                         dparts1[0, :n].reshape(n, 1),
                         dparts1[1, :n].reshape(n, 1), b1r,
                         Wl2, bl2.reshape(1, D), Wr2, br2.reshape(1, D))
    oparts2, dparts2 = sc_layer(xl2, xr2, s, d, att2.reshape(D))

    pooled, h = merge_pool(oparts2[0, :n], oparts2[1, :n],
                           dparts2[0, :n].reshape(n, 1),
                           dparts2[1, :n].reshape(n, 1), b2r,
                           batch.reshape(n, 1))
    return (pooled, h)


# speedup vs baseline: 15.0977x; 1.1698x over previous
import functools

import jax
import jax.numpy as jnp
from jax import lax
from jax.experimental import pallas as pl
from jax.experimental.pallas import tpu as pltpu
from jax.experimental.pallas import tpu_sc as plsc

NC = 2
NS = 16
L = 16
NW = NC * NS
EB = 80
D = 128


def _mm2_body(x_ref, wl_ref, bl_ref, wr_ref, br_ref, xl_ref, xr_ref):
    x = x_ref[...]
    xl_ref[...] = jnp.dot(x, wl_ref[...], preferred_element_type=jnp.float32) + bl_ref[...]
    xr_ref[...] = jnp.dot(x, wr_ref[...], preferred_element_type=jnp.float32) + br_ref[...]


def _merge_mm2_body(pa_ref, pb_ref, da_ref, db_ref, b_ref, wl_ref, bl_ref,
                    wr_ref, br_ref, xl_ref, xr_ref):
    dn = da_ref[...] + db_ref[...] + 1e-16
    h = jnp.maximum((pa_ref[...] + pb_ref[...]) / dn + b_ref[...], 0.0)
    xl_ref[...] = jnp.dot(h, wl_ref[...], preferred_element_type=jnp.float32) + bl_ref[...]
    xr_ref[...] = jnp.dot(h, wr_ref[...], preferred_element_type=jnp.float32) + br_ref[...]


def _make_mm2(n, rb):
    grid = n // rb
    full = lambda i: (0, 0)
    row = lambda i: (i, 0)
    return pl.pallas_call(
        _mm2_body,
        grid=(grid,),
        in_specs=[
            pl.BlockSpec((rb, D), row),
            pl.BlockSpec((D, D), full),
            pl.BlockSpec((1, D), full),
            pl.BlockSpec((D, D), full),
            pl.BlockSpec((1, D), full),
        ],
        out_specs=[pl.BlockSpec((rb, D), row), pl.BlockSpec((rb, D), row)],
        out_shape=[
            jax.ShapeDtypeStruct((n, D), jnp.float32),
            jax.ShapeDtypeStruct((n, D), jnp.float32),
        ],
    )


def _make_merge_mm2(n, rb):
    grid = n // rb
    full = lambda i: (0, 0)
    row = lambda i: (i, 0)
    return pl.pallas_call(
        _merge_mm2_body,
        grid=(grid,),
        in_specs=[
            pl.BlockSpec((rb, D), row),
            pl.BlockSpec((rb, D), row),
            pl.BlockSpec((rb, 1), row),
            pl.BlockSpec((rb, 1), row),
            pl.BlockSpec((1, D), full),
            pl.BlockSpec((D, D), full),
            pl.BlockSpec((1, D), full),
            pl.BlockSpec((D, D), full),
            pl.BlockSpec((1, D), full),
        ],
        out_specs=[pl.BlockSpec((rb, D), row), pl.BlockSpec((rb, D), row)],
        out_shape=[
            jax.ShapeDtypeStruct((n, D), jnp.float32),
            jax.ShapeDtypeStruct((n, D), jnp.float32),
        ],
    )


def _make_merge_pool(n, rb, ng):
    grid = n // rb

    def body(pa_ref, pb_ref, da_ref, db_ref, b_ref, bt_ref, pooled_ref, h_ref,
             acc, cnt):
        i = pl.program_id(0)
        dn = da_ref[...] + db_ref[...] + 1e-16
        h = (pa_ref[...] + pb_ref[...]) / dn + b_ref[...]
        h_ref[...] = h
        groups = lax.broadcasted_iota(jnp.int32, (rb, ng), 1)
        oh = (bt_ref[...] == groups).astype(jnp.float32)
        dn = (((0,), (0,)), ((), ()))
        ps = lax.dot_general(oh, h, dn, preferred_element_type=jnp.float32)
        cs = lax.dot_general(oh, jnp.ones((rb, D), jnp.float32), dn,
                             preferred_element_type=jnp.float32)

        @pl.when(i == 0)
        def _():
            acc[...] = ps
            cnt[...] = cs

        @pl.when(i > 0)
        def _():
            acc[...] = acc[...] + ps
            cnt[...] = cnt[...] + cs

        @pl.when(i == grid - 1)
        def _():
            pooled_ref[...] = acc[...] / jnp.maximum(cnt[...], 1.0)

    full = lambda i: (0, 0)
    row = lambda i: (i, 0)
    return pl.pallas_call(
        body,
        grid=(grid,),
        in_specs=[
            pl.BlockSpec((rb, D), row),
            pl.BlockSpec((rb, D), row),
            pl.BlockSpec((rb, 1), row),
            pl.BlockSpec((rb, 1), row),
            pl.BlockSpec((1, D), full),
            pl.BlockSpec((rb, 1), row),
        ],
        out_specs=[pl.BlockSpec((ng, D), full), pl.BlockSpec((rb, D), row)],
        out_shape=[
            jax.ShapeDtypeStruct((ng, D), jnp.float32),
            jax.ShapeDtypeStruct((n, D), jnp.float32),
        ],
        scratch_shapes=[
            pltpu.VMEM((ng, D), jnp.float32),
            pltpu.VMEM((ng, D), jnp.float32),
        ],
    )


def _leaky(t):
    return jnp.maximum(t, 0.0) + 0.2 * jnp.minimum(t, 0.0)


@functools.lru_cache(maxsize=None)
def _make_sc_layer(etot, epad, np_):
    nblk = epad // (NW * EB)
    assert nblk % 2 == 0
    epw = nblk * EB
    slc = np_ // NS
    zr = 16
    mesh = plsc.VectorSubcoreMesh(core_axis_name="c", subcore_axis_name="s",
                                  num_cores=NC, num_subcores=NS)

    @functools.partial(
        pl.kernel,
        out_type=[
            jax.ShapeDtypeStruct((NC, np_, D), jnp.float32),
            jax.ShapeDtypeStruct((NC, np_), jnp.float32),
        ],
        mesh=mesh,
        compiler_params=pltpu.CompilerParams(needs_layout_passes=False),
        scratch_types=[
            pltpu.VMEM((D,), jnp.float32),
            pltpu.VMEM((EB,), jnp.int32),
            pltpu.VMEM((EB,), jnp.int32),
            pltpu.VMEM((EB, D), jnp.float32),
            pltpu.VMEM((EB, D), jnp.float32),
            pltpu.VMEM((EB,), jnp.float32),
            pltpu.VMEM((EB,), jnp.int32),
            pltpu.VMEM((EB,), jnp.int32),
            pltpu.VMEM((EB, D), jnp.float32),
            pltpu.VMEM((EB, D), jnp.float32),
            pltpu.VMEM((EB,), jnp.float32),
            pltpu.VMEM((zr, D), jnp.float32),
            pltpu.VMEM((slc,), jnp.float32),
            pltpu.VMEM_SHARED((np_, D), jnp.float32),
            pltpu.VMEM_SHARED((np_,), jnp.float32),
            pltpu.SemaphoreType.DMA,
            pltpu.SemaphoreType.DMA,
            pltpu.SemaphoreType.DMA,
            pltpu.SemaphoreType.DMA,
        ],
    )
    def fused(xl_hbm, xr_hbm, s_hbm, d_hbm, att_hbm, out_hbm, dparts_hbm,
              att_v, sidx0, didx0, rows_s0, rows_d0, eev0,
              sidx1, didx1, rows_s1, rows_d1, eev1, zb, zb1, osh, dsh,
              sem_s0, sem_d0, sem_s1, sem_d1):
        c = lax.axis_index("c")
        tid = lax.axis_index("s")
        wid = tid * NC + c

        def zrow(i, _):
            for q in range(D // L):
                zb[i, pl.ds(q * L, L)] = jnp.zeros((L,), jnp.float32)
            return 0
        lax.fori_loop(0, zr, zrow, 0)

        def z1(i, _):
            zb1[pl.ds(i * L, L)] = jnp.zeros((L,), jnp.float32)
            return 0
        lax.fori_loop(0, slc // L, z1, 0)

        def zcp(i, _):
            pltpu.sync_copy(zb, osh.at[pl.ds(tid * slc + i * zr, zr), :])
            return 0
        lax.fori_loop(0, slc // zr, zcp, 0)
        pltpu.sync_copy(zb1, dsh.at[pl.ds(tid * slc, slc)])
        pltpu.sync_copy(att_hbm, att_v)
        plsc.subcore_barrier()

        base = wid * epw
        lanes = lax.iota(jnp.int32, L)
        attv = [att_v[pl.ds(q * L, L)] for q in range(D // L)]

        def fetch(b, sidx, didx, rows_s, rows_d, sem_s, sem_d):
            off = pl.multiple_of(base + b * EB, EB)
            pltpu.sync_copy(s_hbm.at[pl.ds(off, EB)], sidx)
            pltpu.sync_copy(d_hbm.at[pl.ds(off, EB)], didx)
            pltpu.async_copy(xl_hbm.at[sidx], rows_s, sem_s)
            pltpu.async_copy(xr_hbm.at[didx], rows_d, sem_d)

        def drain(sidx, didx, rows_s, rows_d, sem_s, sem_d):
            pltpu.make_async_copy(xl_hbm.at[sidx], rows_s, sem_s).wait()
            pltpu.make_async_copy(xr_hbm.at[didx], rows_d, sem_d).wait()

        def compute(b, sidx, didx, rows_s, rows_d, eev):
            off = pl.multiple_of(base + b * EB, EB)

            def sub(t, _):
                evec = jnp.zeros((L,), jnp.float32)
                for jj in range(L):
                    j = t * L + jj
                    acc = jnp.zeros((L,), jnp.float32)
                    for q in range(D // L):
                        sl = pl.ds(q * L, L)
                        acc = acc + attv[q] * _leaky(rows_s[j, sl] + rows_d[j, sl])
                    evec = jnp.where(lanes == jj, jnp.sum(acc), evec)
                gid = off + t * L + lanes
                ee = jnp.where(gid < etot, jnp.exp(evec), 0.0)
                eev[pl.ds(t * L, L)] = ee
                for jj in range(L):
                    a = ee[jj]
                    j = t * L + jj
                    for q in range(D // L):
                        sl = pl.ds(q * L, L)
                        rows_s[j, sl] = rows_s[j, sl] * a
                return 0
            lax.fori_loop(0, EB // L, sub, 0)

            pltpu.sync_copy(eev, dsh.at[didx], add=True)
            pltpu.sync_copy(rows_s, osh.at[didx], add=True)

        fetch(0, sidx0, didx0, rows_s0, rows_d0, sem_s0, sem_d0)

        def blk2(ii, _):
            b0 = 2 * ii
            fetch(b0 + 1, sidx1, didx1, rows_s1, rows_d1, sem_s1, sem_d1)
            drain(sidx0, didx0, rows_s0, rows_d0, sem_s0, sem_d0)
            compute(b0, sidx0, didx0, rows_s0, rows_d0, eev0)
            fetch(b0 + 2, sidx0, didx0, rows_s0, rows_d0, sem_s0, sem_d0)
            drain(sidx1, didx1, rows_s1, rows_d1, sem_s1, sem_d1)
            compute(b0 + 1, sidx1, didx1, rows_s1, rows_d1, eev1)
            return 0
        lax.fori_loop(0, nblk // 2, blk2, 0)
        drain(sidx0, didx0, rows_s0, rows_d0, sem_s0, sem_d0)

        plsc.subcore_barrier()
        pltpu.sync_copy(osh.at[pl.ds(tid * slc, slc), :],
                        out_hbm.at[c, pl.ds(tid * slc, slc), :])
        pltpu.sync_copy(dsh.at[pl.ds(tid * slc, slc)],
                        dparts_hbm.at[c, pl.ds(tid * slc, slc)])

    return fused


def kernel(x, edge_index, batch, Wl1, bl1, Wr1, br1, att1, b1,
           Wl2, bl2, Wr2, br2, att2, b2):
    n = x.shape[0]
    e = edge_index.shape[1]
    etot = e + n
    nblk_w = (etot + NW * EB - 1) // (NW * EB)
    nblk_w += nblk_w % 2
    epad = nblk_w * NW * EB
    np_ = ((n + NS * 128 - 1) // (NS * 128)) * (NS * 128)
    ng = 16

    loop = jnp.arange(n, dtype=jnp.int32)
    zpad = jnp.zeros((epad - etot + EB,), jnp.int32)
    s = jnp.concatenate([edge_index[0], loop, zpad])
    d = jnp.concatenate([edge_index[1], loop, zpad])

    sc_layer = _make_sc_layer(etot, epad, np_)
    mm2 = _make_mm2(n, 1000)
    merge_mm2 = _make_merge_mm2(n, 1000)
    merge_pool = _make_merge_pool(n, 1000, ng)

    b1r = b1.reshape(1, D)
    b2r = b2.reshape(1, D)

    xl1, xr1 = mm2(x, Wl1, bl1.reshape(1, D), Wr1, br1.reshape(1, D))
    oparts1, dparts1 = sc_layer(xl1, xr1, s, d, att1.reshape(D))

    xl2, xr2 = merge_mm2(oparts1[0, :n], oparts1[1, :n],
                         dparts1[0, :n].reshape(n, 1),
                         dparts1[1, :n].reshape(n, 1), b1r,
                         Wl2, bl2.reshape(1, D), Wr2, br2.reshape(1, D))
    oparts2, dparts2 = sc_layer(xl2, xr2, s, d, att2.reshape(D))

    pooled, h = merge_pool(oparts2[0, :n], oparts2[1, :n],
                           dparts2[0, :n].reshape(n, 1),
                           dparts2[1, :n].reshape(n, 1), b2r,
                           batch.reshape(n, 1))
    return (pooled, h)


# speedup vs baseline: 16.0091x; 1.0604x over previous
import functools

import jax
import jax.numpy as jnp
from jax import lax
from jax.experimental import pallas as pl
from jax.experimental.pallas import tpu as pltpu
from jax.experimental.pallas import tpu_sc as plsc

NC = 2
NS = 16
L = 16
NW = NC * NS
EB = 80
D = 128


def _mm2_body(x_ref, wl_ref, bl_ref, wr_ref, br_ref, xl_ref, xr_ref):
    x = x_ref[...]
    xl_ref[...] = jnp.dot(x, wl_ref[...], preferred_element_type=jnp.float32) + bl_ref[...]
    xr_ref[...] = jnp.dot(x, wr_ref[...], preferred_element_type=jnp.float32) + br_ref[...]


def _merge_mm2_body(pa_ref, pb_ref, da_ref, db_ref, b_ref, wl_ref, bl_ref,
                    wr_ref, br_ref, xl_ref, xr_ref):
    dn = da_ref[...] + db_ref[...] + 1e-16
    h = jnp.maximum((pa_ref[...] + pb_ref[...]) / dn + b_ref[...], 0.0)
    xl_ref[...] = jnp.dot(h, wl_ref[...], preferred_element_type=jnp.float32) + bl_ref[...]
    xr_ref[...] = jnp.dot(h, wr_ref[...], preferred_element_type=jnp.float32) + br_ref[...]


def _make_mm2(n, rb):
    grid = n // rb
    full = lambda i: (0, 0)
    row = lambda i: (i, 0)
    return pl.pallas_call(
        _mm2_body,
        grid=(grid,),
        in_specs=[
            pl.BlockSpec((rb, D), row),
            pl.BlockSpec((D, D), full),
            pl.BlockSpec((1, D), full),
            pl.BlockSpec((D, D), full),
            pl.BlockSpec((1, D), full),
        ],
        out_specs=[pl.BlockSpec((rb, D), row), pl.BlockSpec((rb, D), row)],
        out_shape=[
            jax.ShapeDtypeStruct((n, D), jnp.float32),
            jax.ShapeDtypeStruct((n, D), jnp.float32),
        ],
    )


def _make_merge_mm2(n, rb):
    grid = n // rb
    full = lambda i: (0, 0)
    row = lambda i: (i, 0)
    return pl.pallas_call(
        _merge_mm2_body,
        grid=(grid,),
        in_specs=[
            pl.BlockSpec((rb, D), row),
            pl.BlockSpec((rb, D), row),
            pl.BlockSpec((rb, 1), row),
            pl.BlockSpec((rb, 1), row),
            pl.BlockSpec((1, D), full),
            pl.BlockSpec((D, D), full),
            pl.BlockSpec((1, D), full),
            pl.BlockSpec((D, D), full),
            pl.BlockSpec((1, D), full),
        ],
        out_specs=[pl.BlockSpec((rb, D), row), pl.BlockSpec((rb, D), row)],
        out_shape=[
            jax.ShapeDtypeStruct((n, D), jnp.float32),
            jax.ShapeDtypeStruct((n, D), jnp.float32),
        ],
    )


def _make_merge_pool(n, rb, ng):
    grid = n // rb

    def body(pa_ref, pb_ref, da_ref, db_ref, b_ref, bt_ref, pooled_ref, h_ref,
             acc, cnt):
        i = pl.program_id(0)
        dn = da_ref[...] + db_ref[...] + 1e-16
        h = (pa_ref[...] + pb_ref[...]) / dn + b_ref[...]
        h_ref[...] = h
        groups = lax.broadcasted_iota(jnp.int32, (rb, ng), 1)
        oh = (bt_ref[...] == groups).astype(jnp.float32)
        dn = (((0,), (0,)), ((), ()))
        ps = lax.dot_general(oh, h, dn, preferred_element_type=jnp.float32)
        cs = lax.dot_general(oh, jnp.ones((rb, D), jnp.float32), dn,
                             preferred_element_type=jnp.float32)

        @pl.when(i == 0)
        def _():
            acc[...] = ps
            cnt[...] = cs

        @pl.when(i > 0)
        def _():
            acc[...] = acc[...] + ps
            cnt[...] = cnt[...] + cs

        @pl.when(i == grid - 1)
        def _():
            pooled_ref[...] = acc[...] / jnp.maximum(cnt[...], 1.0)

    full = lambda i: (0, 0)
    row = lambda i: (i, 0)
    return pl.pallas_call(
        body,
        grid=(grid,),
        in_specs=[
            pl.BlockSpec((rb, D), row),
            pl.BlockSpec((rb, D), row),
            pl.BlockSpec((rb, 1), row),
            pl.BlockSpec((rb, 1), row),
            pl.BlockSpec((1, D), full),
            pl.BlockSpec((rb, 1), row),
        ],
        out_specs=[pl.BlockSpec((ng, D), full), pl.BlockSpec((rb, D), row)],
        out_shape=[
            jax.ShapeDtypeStruct((ng, D), jnp.float32),
            jax.ShapeDtypeStruct((n, D), jnp.float32),
        ],
        scratch_shapes=[
            pltpu.VMEM((ng, D), jnp.float32),
            pltpu.VMEM((ng, D), jnp.float32),
        ],
    )


def _leaky(t):
    return jnp.maximum(t, 0.0) + 0.2 * jnp.minimum(t, 0.0)


@functools.lru_cache(maxsize=None)
def _make_sc_layer(etot, epad, np_):
    nblk = epad // (NW * EB)
    assert nblk % 2 == 0
    epw = nblk * EB
    slc = np_ // NS
    zr = 16
    mesh = plsc.VectorSubcoreMesh(core_axis_name="c", subcore_axis_name="s",
                                  num_cores=NC, num_subcores=NS)

    @functools.partial(
        pl.kernel,
        out_type=[
            jax.ShapeDtypeStruct((NC, np_, D), jnp.float32),
            jax.ShapeDtypeStruct((NC, np_), jnp.float32),
        ],
        mesh=mesh,
        compiler_params=pltpu.CompilerParams(needs_layout_passes=False),
        scratch_types=[
            pltpu.VMEM((D,), jnp.float32),
            pltpu.VMEM((EB,), jnp.int32),
            pltpu.VMEM((EB,), jnp.int32),
            pltpu.VMEM((EB, D), jnp.float32),
            pltpu.VMEM((EB, D), jnp.float32),
            pltpu.VMEM((EB,), jnp.float32),
            pltpu.VMEM((EB,), jnp.int32),
            pltpu.VMEM((EB,), jnp.int32),
            pltpu.VMEM((EB, D), jnp.float32),
            pltpu.VMEM((EB, D), jnp.float32),
            pltpu.VMEM((EB,), jnp.float32),
            pltpu.VMEM((zr, D), jnp.float32),
            pltpu.VMEM((slc,), jnp.float32),
            pltpu.VMEM_SHARED((np_, D), jnp.float32),
            pltpu.VMEM_SHARED((np_,), jnp.float32),
            pltpu.SemaphoreType.DMA,
            pltpu.SemaphoreType.DMA,
            pltpu.SemaphoreType.DMA,
            pltpu.SemaphoreType.DMA,
        ],
    )
    def fused(xl_hbm, xr_hbm, s_hbm, d_hbm, att_hbm, out_hbm, dparts_hbm,
              att_v, sidx0, didx0, rows_s0, rows_d0, eev0,
              sidx1, didx1, rows_s1, rows_d1, eev1, zb, zb1, osh, dsh,
              sem_s0, sem_d0, sem_s1, sem_d1):
        c = lax.axis_index("c")
        tid = lax.axis_index("s")
        wid = tid * NC + c

        def zrow(i, _):
            for q in range(D // L):
                zb[i, pl.ds(q * L, L)] = jnp.zeros((L,), jnp.float32)
            return 0
        lax.fori_loop(0, zr, zrow, 0)

        def z1(i, _):
            zb1[pl.ds(i * L, L)] = jnp.zeros((L,), jnp.float32)
            return 0
        lax.fori_loop(0, slc // L, z1, 0)

        def zcp(i, _):
            pltpu.sync_copy(zb, osh.at[pl.ds(tid * slc + i * zr, zr), :])
            return 0
        lax.fori_loop(0, slc // zr, zcp, 0)
        pltpu.sync_copy(zb1, dsh.at[pl.ds(tid * slc, slc)])
        pltpu.sync_copy(att_hbm, att_v)
        plsc.subcore_barrier()

        base = wid * epw
        lanes = lax.iota(jnp.int32, L)
        attv = [att_v[pl.ds(q * L, L)] for q in range(D // L)]

        def fetch(b, sidx, didx, rows_s, rows_d, sem_s, sem_d):
            off = pl.multiple_of(base + b * EB, EB)
            pltpu.sync_copy(s_hbm.at[pl.ds(off, EB)], sidx)
            pltpu.sync_copy(d_hbm.at[pl.ds(off, EB)], didx)
            pltpu.async_copy(xl_hbm.at[sidx], rows_s, sem_s)
            pltpu.async_copy(xr_hbm.at[didx], rows_d, sem_d)

        def drain(sidx, didx, rows_s, rows_d, sem_s, sem_d):
            pltpu.make_async_copy(xl_hbm.at[sidx], rows_s, sem_s).wait()
            pltpu.make_async_copy(xr_hbm.at[didx], rows_d, sem_d).wait()

        def compute(b, sidx, didx, rows_s, rows_d, eev):
            off = pl.multiple_of(base + b * EB, EB)

            def sub(t, _):
                evec = jnp.zeros((L,), jnp.float32)
                for jj in range(L):
                    j = t * L + jj
                    vs = [rows_s[j, pl.ds(q * L, L)] for q in range(D // L)]
                    acc = jnp.zeros((L,), jnp.float32)
                    for q in range(D // L):
                        tt = vs[q] + rows_d[j, pl.ds(q * L, L)]
                        acc = acc + attv[q] * jnp.maximum(tt, 0.2 * tt)
                    ee = jnp.where(off + j < etot,
                                   jnp.exp(jnp.broadcast_to(jnp.sum(acc), (L,))),
                                   0.0)
                    evec = jnp.where(lanes == jj, ee, evec)
                    for q in range(D // L):
                        rows_s[j, pl.ds(q * L, L)] = vs[q] * ee
                eev[pl.ds(t * L, L)] = evec
                return 0
            lax.fori_loop(0, EB // L, sub, 0)

            pltpu.sync_copy(eev, dsh.at[didx], add=True)
            pltpu.sync_copy(rows_s, osh.at[didx], add=True)

        fetch(0, sidx0, didx0, rows_s0, rows_d0, sem_s0, sem_d0)

        def blk2(ii, _):
            b0 = 2 * ii
            fetch(b0 + 1, sidx1, didx1, rows_s1, rows_d1, sem_s1, sem_d1)
            drain(sidx0, didx0, rows_s0, rows_d0, sem_s0, sem_d0)
            compute(b0, sidx0, didx0, rows_s0, rows_d0, eev0)
            fetch(b0 + 2, sidx0, didx0, rows_s0, rows_d0, sem_s0, sem_d0)
            drain(sidx1, didx1, rows_s1, rows_d1, sem_s1, sem_d1)
            compute(b0 + 1, sidx1, didx1, rows_s1, rows_d1, eev1)
            return 0
        lax.fori_loop(0, nblk // 2, blk2, 0)
        drain(sidx0, didx0, rows_s0, rows_d0, sem_s0, sem_d0)

        plsc.subcore_barrier()
        pltpu.sync_copy(osh.at[pl.ds(tid * slc, slc), :],
                        out_hbm.at[c, pl.ds(tid * slc, slc), :])
        pltpu.sync_copy(dsh.at[pl.ds(tid * slc, slc)],
                        dparts_hbm.at[c, pl.ds(tid * slc, slc)])

    return fused


def kernel(x, edge_index, batch, Wl1, bl1, Wr1, br1, att1, b1,
           Wl2, bl2, Wr2, br2, att2, b2):
    n = x.shape[0]
    e = edge_index.shape[1]
    etot = e + n
    nblk_w = (etot + NW * EB - 1) // (NW * EB)
    nblk_w += nblk_w % 2
    epad = nblk_w * NW * EB
    np_ = ((n + NS * 128 - 1) // (NS * 128)) * (NS * 128)
    ng = 16

    loop = jnp.arange(n, dtype=jnp.int32)
    zpad = jnp.zeros((epad - etot + EB,), jnp.int32)
    s = jnp.concatenate([edge_index[0], loop, zpad])
    d = jnp.concatenate([edge_index[1], loop, zpad])

    sc_layer = _make_sc_layer(etot, epad, np_)
    mm2 = _make_mm2(n, 1000)
    merge_mm2 = _make_merge_mm2(n, 1000)
    merge_pool = _make_merge_pool(n, 1000, ng)

    b1r = b1.reshape(1, D)
    b2r = b2.reshape(1, D)

    xl1, xr1 = mm2(x, Wl1, bl1.reshape(1, D), Wr1, br1.reshape(1, D))
    oparts1, dparts1 = sc_layer(xl1, xr1, s, d, att1.reshape(D))

    xl2, xr2 = merge_mm2(oparts1[0, :n], oparts1[1, :n],
                         dparts1[0, :n].reshape(n, 1),
                         dparts1[1, :n].reshape(n, 1), b1r,
                         Wl2, bl2.reshape(1, D), Wr2, br2.reshape(1, D))
    oparts2, dparts2 = sc_layer(xl2, xr2, s, d, att2.reshape(D))

    pooled, h = merge_pool(oparts2[0, :n], oparts2[1, :n],
                           dparts2[0, :n].reshape(n, 1),
                           dparts2[1, :n].reshape(n, 1), b2r,
                           batch.reshape(n, 1))
    return (pooled, h)


# speedup vs baseline: 16.8138x; 1.0503x over previous
import functools

import jax
import jax.numpy as jnp
from jax import lax
from jax.experimental import pallas as pl
from jax.experimental.pallas import tpu as pltpu
from jax.experimental.pallas import tpu_sc as plsc

NC = 2
NS = 16
L = 16
NW = NC * NS
EB = 80
D = 128


def _mm2_body(x_ref, wl_ref, bl_ref, wr_ref, br_ref, xl_ref, xr_ref):
    x = x_ref[...]
    xl_ref[...] = jnp.dot(x, wl_ref[...], preferred_element_type=jnp.float32) + bl_ref[...]
    xr_ref[...] = jnp.dot(x, wr_ref[...], preferred_element_type=jnp.float32) + br_ref[...]


def _merge_mm2_body(pa_ref, pb_ref, da_ref, db_ref, b_ref, wl_ref, bl_ref,
                    wr_ref, br_ref, xl_ref, xr_ref):
    dn = da_ref[...] + db_ref[...] + 1e-16
    h = jnp.maximum((pa_ref[...] + pb_ref[...]) / dn + b_ref[...], 0.0)
    xl_ref[...] = jnp.dot(h, wl_ref[...], preferred_element_type=jnp.float32) + bl_ref[...]
    xr_ref[...] = jnp.dot(h, wr_ref[...], preferred_element_type=jnp.float32) + br_ref[...]


def _make_mm2(n, rb):
    grid = n // rb
    full = lambda i: (0, 0)
    row = lambda i: (i, 0)
    return pl.pallas_call(
        _mm2_body,
        grid=(grid,),
        in_specs=[
            pl.BlockSpec((rb, D), row),
            pl.BlockSpec((D, D), full),
            pl.BlockSpec((1, D), full),
            pl.BlockSpec((D, D), full),
            pl.BlockSpec((1, D), full),
        ],
        out_specs=[pl.BlockSpec((rb, D), row), pl.BlockSpec((rb, D), row)],
        out_shape=[
            jax.ShapeDtypeStruct((n, D), jnp.float32),
            jax.ShapeDtypeStruct((n, D), jnp.float32),
        ],
    )


def _make_merge_mm2(n, rb):
    grid = n // rb
    full = lambda i: (0, 0)
    row = lambda i: (i, 0)
    return pl.pallas_call(
        _merge_mm2_body,
        grid=(grid,),
        in_specs=[
            pl.BlockSpec((rb, D), row),
            pl.BlockSpec((rb, D), row),
            pl.BlockSpec((rb, 1), row),
            pl.BlockSpec((rb, 1), row),
            pl.BlockSpec((1, D), full),
            pl.BlockSpec((D, D), full),
            pl.BlockSpec((1, D), full),
            pl.BlockSpec((D, D), full),
            pl.BlockSpec((1, D), full),
        ],
        out_specs=[pl.BlockSpec((rb, D), row), pl.BlockSpec((rb, D), row)],
        out_shape=[
            jax.ShapeDtypeStruct((n, D), jnp.float32),
            jax.ShapeDtypeStruct((n, D), jnp.float32),
        ],
    )


def _make_merge_pool(n, rb, ng):
    grid = n // rb

    def body(pa_ref, pb_ref, da_ref, db_ref, b_ref, bt_ref, pooled_ref, h_ref,
             acc, cnt):
        i = pl.program_id(0)
        dn = da_ref[...] + db_ref[...] + 1e-16
        h = (pa_ref[...] + pb_ref[...]) / dn + b_ref[...]
        h_ref[...] = h
        groups = lax.broadcasted_iota(jnp.int32, (rb, ng), 1)
        oh = (bt_ref[...] == groups).astype(jnp.float32)
        dn = (((0,), (0,)), ((), ()))
        ps = lax.dot_general(oh, h, dn, preferred_element_type=jnp.float32)
        cs = lax.dot_general(oh, jnp.ones((rb, D), jnp.float32), dn,
                             preferred_element_type=jnp.float32)

        @pl.when(i == 0)
        def _():
            acc[...] = ps
            cnt[...] = cs

        @pl.when(i > 0)
        def _():
            acc[...] = acc[...] + ps
            cnt[...] = cnt[...] + cs

        @pl.when(i == grid - 1)
        def _():
            pooled_ref[...] = acc[...] / jnp.maximum(cnt[...], 1.0)

    full = lambda i: (0, 0)
    row = lambda i: (i, 0)
    return pl.pallas_call(
        body,
        grid=(grid,),
        in_specs=[
            pl.BlockSpec((rb, D), row),
            pl.BlockSpec((rb, D), row),
            pl.BlockSpec((rb, 1), row),
            pl.BlockSpec((rb, 1), row),
            pl.BlockSpec((1, D), full),
            pl.BlockSpec((rb, 1), row),
        ],
        out_specs=[pl.BlockSpec((ng, D), full), pl.BlockSpec((rb, D), row)],
        out_shape=[
            jax.ShapeDtypeStruct((ng, D), jnp.float32),
            jax.ShapeDtypeStruct((n, D), jnp.float32),
        ],
        scratch_shapes=[
            pltpu.VMEM((ng, D), jnp.float32),
            pltpu.VMEM((ng, D), jnp.float32),
        ],
    )


def _leaky(t):
    return jnp.maximum(t, 0.0) + 0.2 * jnp.minimum(t, 0.0)


@functools.lru_cache(maxsize=None)
def _make_sc_layer(etot, epad, np_):
    nblk = epad // (NW * EB)
    assert nblk % 2 == 0
    epw = nblk * EB
    slc = np_ // NS
    zr = 16
    mesh = plsc.VectorSubcoreMesh(core_axis_name="c", subcore_axis_name="s",
                                  num_cores=NC, num_subcores=NS)

    @functools.partial(
        pl.kernel,
        out_type=[
            jax.ShapeDtypeStruct((NC, np_, D), jnp.float32),
            jax.ShapeDtypeStruct((NC, np_), jnp.float32),
        ],
        mesh=mesh,
        compiler_params=pltpu.CompilerParams(needs_layout_passes=False),
        scratch_types=[
            pltpu.VMEM((D,), jnp.float32),
            pltpu.VMEM((2, EB), jnp.int32),
            pltpu.VMEM((EB, D), jnp.float32),
            pltpu.VMEM((EB, D), jnp.float32),
            pltpu.VMEM((EB,), jnp.float32),
            pltpu.VMEM((2, EB), jnp.int32),
            pltpu.VMEM((EB, D), jnp.float32),
            pltpu.VMEM((EB, D), jnp.float32),
            pltpu.VMEM((EB,), jnp.float32),
            pltpu.VMEM((zr, D), jnp.float32),
            pltpu.VMEM((slc,), jnp.float32),
            pltpu.VMEM_SHARED((np_, D), jnp.float32),
            pltpu.VMEM_SHARED((np_,), jnp.float32),
            pltpu.SemaphoreType.DMA,
            pltpu.SemaphoreType.DMA,
            pltpu.SemaphoreType.DMA,
            pltpu.SemaphoreType.DMA,
        ],
    )
    def fused(xl_hbm, xr_hbm, sd_hbm, att_hbm, out_hbm, dparts_hbm,
              att_v, idx0, rows_s0, rows_d0, eev0,
              idx1, rows_s1, rows_d1, eev1, zb, zb1, osh, dsh,
              sem_s0, sem_d0, sem_s1, sem_d1):
        c = lax.axis_index("c")
        tid = lax.axis_index("s")
        wid = tid * NC + c

        def zrow(i, _):
            for q in range(D // L):
                zb[i, pl.ds(q * L, L)] = jnp.zeros((L,), jnp.float32)
            return 0
        lax.fori_loop(0, zr, zrow, 0)

        def z1(i, _):
            zb1[pl.ds(i * L, L)] = jnp.zeros((L,), jnp.float32)
            return 0
        lax.fori_loop(0, slc // L, z1, 0)

        def zcp(i, _):
            pltpu.sync_copy(zb, osh.at[pl.ds(tid * slc + i * zr, zr), :])
            return 0
        lax.fori_loop(0, slc // zr, zcp, 0)
        pltpu.sync_copy(zb1, dsh.at[pl.ds(tid * slc, slc)])
        pltpu.sync_copy(att_hbm, att_v)
        plsc.subcore_barrier()

        base = wid * epw
        brow = wid * nblk
        lanes = lax.iota(jnp.int32, L)
        attv = [att_v[pl.ds(q * L, L)] for q in range(D // L)]

        def fetch(b, idx, rows_s, rows_d, sem_s, sem_d):
            pltpu.sync_copy(sd_hbm.at[brow + b], idx)
            pltpu.async_copy(xl_hbm.at[idx.at[0]], rows_s, sem_s)
            pltpu.async_copy(xr_hbm.at[idx.at[1]], rows_d, sem_d)

        def drain(idx, rows_s, rows_d, sem_s, sem_d):
            pltpu.make_async_copy(xl_hbm.at[idx.at[0]], rows_s, sem_s).wait()
            pltpu.make_async_copy(xr_hbm.at[idx.at[1]], rows_d, sem_d).wait()

        def compute(b, idx, rows_s, rows_d, eev):
            off = pl.multiple_of(base + b * EB, EB)

            def sub(t, _):
                evec = jnp.zeros((L,), jnp.float32)
                for jj in range(L):
                    j = t * L + jj
                    vs = [rows_s[j, pl.ds(q * L, L)] for q in range(D // L)]
                    acc = jnp.zeros((L,), jnp.float32)
                    for q in range(D // L):
                        tt = vs[q] + rows_d[j, pl.ds(q * L, L)]
                        acc = acc + attv[q] * jnp.maximum(tt, 0.2 * tt)
                    ee = jnp.where(off + j < etot,
                                   jnp.exp(jnp.broadcast_to(jnp.sum(acc), (L,))),
                                   0.0)
                    evec = jnp.where(lanes == jj, ee, evec)
                    for q in range(D // L):
                        rows_s[j, pl.ds(q * L, L)] = vs[q] * ee
                eev[pl.ds(t * L, L)] = evec
                return 0
            lax.fori_loop(0, EB // L, sub, 0)

            pltpu.sync_copy(eev, dsh.at[idx.at[1]], add=True)
            pltpu.sync_copy(rows_s, osh.at[idx.at[1]], add=True)

        fetch(0, idx0, rows_s0, rows_d0, sem_s0, sem_d0)

        def blk2(ii, _):
            b0 = 2 * ii
            fetch(b0 + 1, idx1, rows_s1, rows_d1, sem_s1, sem_d1)
            drain(idx0, rows_s0, rows_d0, sem_s0, sem_d0)
            compute(b0, idx0, rows_s0, rows_d0, eev0)
            fetch(b0 + 2, idx0, rows_s0, rows_d0, sem_s0, sem_d0)
            drain(idx1, rows_s1, rows_d1, sem_s1, sem_d1)
            compute(b0 + 1, idx1, rows_s1, rows_d1, eev1)
            return 0
        lax.fori_loop(0, nblk // 2, blk2, 0)
        drain(idx0, rows_s0, rows_d0, sem_s0, sem_d0)

        plsc.subcore_barrier()
        pltpu.sync_copy(osh.at[pl.ds(tid * slc, slc), :],
                        out_hbm.at[c, pl.ds(tid * slc, slc), :])
        pltpu.sync_copy(dsh.at[pl.ds(tid * slc, slc)],
                        dparts_hbm.at[c, pl.ds(tid * slc, slc)])

    return fused


def kernel(x, edge_index, batch, Wl1, bl1, Wr1, br1, att1, b1,
           Wl2, bl2, Wr2, br2, att2, b2):
    n = x.shape[0]
    e = edge_index.shape[1]
    etot = e + n
    nblk_w = (etot + NW * EB - 1) // (NW * EB)
    nblk_w += nblk_w % 2
    epad = nblk_w * NW * EB
    np_ = ((n + NS * 128 - 1) // (NS * 128)) * (NS * 128)
    ng = 16

    loop = jnp.arange(n, dtype=jnp.int32)
    zpad = jnp.zeros((epad - etot + EB,), jnp.int32)
    s = jnp.concatenate([edge_index[0], loop, zpad])
    d = jnp.concatenate([edge_index[1], loop, zpad])
    sd = jnp.stack([s.reshape(-1, EB), d.reshape(-1, EB)], axis=1)

    sc_layer = _make_sc_layer(etot, epad, np_)
    mm2 = _make_mm2(n, 1000)
    merge_mm2 = _make_merge_mm2(n, 1000)
    merge_pool = _make_merge_pool(n, 1000, ng)

    b1r = b1.reshape(1, D)
    b2r = b2.reshape(1, D)

    xl1, xr1 = mm2(x, Wl1, bl1.reshape(1, D), Wr1, br1.reshape(1, D))
    oparts1, dparts1 = sc_layer(xl1, xr1, sd, att1.reshape(D))

    xl2, xr2 = merge_mm2(oparts1[0, :n], oparts1[1, :n],
                         dparts1[0, :n].reshape(n, 1),
                         dparts1[1, :n].reshape(n, 1), b1r,
                         Wl2, bl2.reshape(1, D), Wr2, br2.reshape(1, D))
    oparts2, dparts2 = sc_layer(xl2, xr2, sd, att2.reshape(D))

    pooled, h = merge_pool(oparts2[0, :n], oparts2[1, :n],
                           dparts2[0, :n].reshape(n, 1),
                           dparts2[1, :n].reshape(n, 1), b2r,
                           batch.reshape(n, 1))
    return (pooled, h)


# speedup vs baseline: 16.9249x; 1.0066x over previous
import functools

import jax
import jax.numpy as jnp
from jax import lax
from jax.experimental import pallas as pl
from jax.experimental.pallas import tpu as pltpu
from jax.experimental.pallas import tpu_sc as plsc

NC = 2
NS = 16
L = 16
NW = NC * NS
EB = 80
D = 128


def _mm2_body(x_ref, wl_ref, bl_ref, wr_ref, br_ref, xl_ref, xr_ref):
    x = x_ref[...]
    xl_ref[...] = jnp.dot(x, wl_ref[...], preferred_element_type=jnp.float32) + bl_ref[...]
    xr_ref[...] = jnp.dot(x, wr_ref[...], preferred_element_type=jnp.float32) + br_ref[...]


def _merge_mm2_body(pa_ref, pb_ref, da_ref, db_ref, b_ref, wl_ref, bl_ref,
                    wr_ref, br_ref, xl_ref, xr_ref):
    dn = da_ref[...] + db_ref[...] + 1e-16
    h = jnp.maximum((pa_ref[...] + pb_ref[...]) / dn + b_ref[...], 0.0)
    xl_ref[...] = jnp.dot(h, wl_ref[...], preferred_element_type=jnp.float32) + bl_ref[...]
    xr_ref[...] = jnp.dot(h, wr_ref[...], preferred_element_type=jnp.float32) + br_ref[...]


def _make_mm2(n, rb):
    grid = n // rb
    full = lambda i: (0, 0)
    row = lambda i: (i, 0)
    return pl.pallas_call(
        _mm2_body,
        grid=(grid,),
        in_specs=[
            pl.BlockSpec((rb, D), row),
            pl.BlockSpec((D, D), full),
            pl.BlockSpec((1, D), full),
            pl.BlockSpec((D, D), full),
            pl.BlockSpec((1, D), full),
        ],
        out_specs=[pl.BlockSpec((rb, D), row), pl.BlockSpec((rb, D), row)],
        out_shape=[
            jax.ShapeDtypeStruct((n, D), jnp.float32),
            jax.ShapeDtypeStruct((n, D), jnp.float32),
        ],
    )


def _make_merge_mm2(n, rb):
    grid = n // rb
    full = lambda i: (0, 0)
    row = lambda i: (i, 0)
    return pl.pallas_call(
        _merge_mm2_body,
        grid=(grid,),
        in_specs=[
            pl.BlockSpec((rb, D), row),
            pl.BlockSpec((rb, D), row),
            pl.BlockSpec((rb, 1), row),
            pl.BlockSpec((rb, 1), row),
            pl.BlockSpec((1, D), full),
            pl.BlockSpec((D, D), full),
            pl.BlockSpec((1, D), full),
            pl.BlockSpec((D, D), full),
            pl.BlockSpec((1, D), full),
        ],
        out_specs=[pl.BlockSpec((rb, D), row), pl.BlockSpec((rb, D), row)],
        out_shape=[
            jax.ShapeDtypeStruct((n, D), jnp.float32),
            jax.ShapeDtypeStruct((n, D), jnp.float32),
        ],
    )


def _make_merge_pool(n, rb, ng):
    grid = n // rb

    def body(pa_ref, pb_ref, da_ref, db_ref, b_ref, bt_ref, pooled_ref, h_ref,
             acc, cnt):
        i = pl.program_id(0)
        dn = da_ref[...] + db_ref[...] + 1e-16
        h = (pa_ref[...] + pb_ref[...]) / dn + b_ref[...]
        h_ref[...] = h
        groups = lax.broadcasted_iota(jnp.int32, (rb, ng), 1)
        oh = (bt_ref[...] == groups).astype(jnp.float32)
        dn = (((0,), (0,)), ((), ()))
        ps = lax.dot_general(oh, h, dn, preferred_element_type=jnp.float32)
        cs = lax.dot_general(oh, jnp.ones((rb, D), jnp.float32), dn,
                             preferred_element_type=jnp.float32)

        @pl.when(i == 0)
        def _():
            acc[...] = ps
            cnt[...] = cs

        @pl.when(i > 0)
        def _():
            acc[...] = acc[...] + ps
            cnt[...] = cnt[...] + cs

        @pl.when(i == grid - 1)
        def _():
            pooled_ref[...] = acc[...] / jnp.maximum(cnt[...], 1.0)

    full = lambda i: (0, 0)
    row = lambda i: (i, 0)
    return pl.pallas_call(
        body,
        grid=(grid,),
        in_specs=[
            pl.BlockSpec((rb, D), row),
            pl.BlockSpec((rb, D), row),
            pl.BlockSpec((rb, 1), row),
            pl.BlockSpec((rb, 1), row),
            pl.BlockSpec((1, D), full),
            pl.BlockSpec((rb, 1), row),
        ],
        out_specs=[pl.BlockSpec((ng, D), full), pl.BlockSpec((rb, D), row)],
        out_shape=[
            jax.ShapeDtypeStruct((ng, D), jnp.float32),
            jax.ShapeDtypeStruct((n, D), jnp.float32),
        ],
        scratch_shapes=[
            pltpu.VMEM((ng, D), jnp.float32),
            pltpu.VMEM((ng, D), jnp.float32),
        ],
    )


def _leaky(t):
    return jnp.maximum(t, 0.0) + 0.2 * jnp.minimum(t, 0.0)


@functools.lru_cache(maxsize=None)
def _make_sc_layer(etot, epad, np_):
    nblk = epad // (NW * EB)
    assert nblk % 2 == 0
    epw = nblk * EB
    slc = np_ // NS
    zr = 16
    mesh = plsc.VectorSubcoreMesh(core_axis_name="c", subcore_axis_name="s",
                                  num_cores=NC, num_subcores=NS)

    @functools.partial(
        pl.kernel,
        out_type=[
            jax.ShapeDtypeStruct((NC, np_, D), jnp.float32),
            jax.ShapeDtypeStruct((NC, np_), jnp.float32),
        ],
        mesh=mesh,
        compiler_params=pltpu.CompilerParams(needs_layout_passes=False),
        scratch_types=[
            pltpu.VMEM((D,), jnp.float32),
            pltpu.VMEM((2, EB), jnp.int32),
            pltpu.VMEM((EB, D), jnp.float32),
            pltpu.VMEM((EB, D), jnp.float32),
            pltpu.VMEM((EB,), jnp.float32),
            pltpu.VMEM((2, EB), jnp.int32),
            pltpu.VMEM((EB, D), jnp.float32),
            pltpu.VMEM((EB, D), jnp.float32),
            pltpu.VMEM((EB,), jnp.float32),
            pltpu.VMEM((zr, D), jnp.float32),
            pltpu.VMEM((slc,), jnp.float32),
            pltpu.VMEM_SHARED((np_, D), jnp.float32),
            pltpu.VMEM_SHARED((np_,), jnp.float32),
            pltpu.SemaphoreType.DMA,
            pltpu.SemaphoreType.DMA,
            pltpu.SemaphoreType.DMA,
            pltpu.SemaphoreType.DMA,
        ],
    )
    def fused(xl_hbm, xr_hbm, sd_hbm, att_hbm, out_hbm, dparts_hbm,
              att_v, idx0, rows_s0, rows_d0, eev0,
              idx1, rows_s1, rows_d1, eev1, zb, zb1, osh, dsh,
              sem_s0, sem_d0, sem_s1, sem_d1):
        c = lax.axis_index("c")
        tid = lax.axis_index("s")
        wid = tid * NC + c

        def zrow(i, _):
            for q in range(D // L):
                zb[i, pl.ds(q * L, L)] = jnp.zeros((L,), jnp.float32)
            return 0
        lax.fori_loop(0, zr, zrow, 0)

        def z1(i, _):
            zb1[pl.ds(i * L, L)] = jnp.zeros((L,), jnp.float32)
            return 0
        lax.fori_loop(0, slc // L, z1, 0)

        def zcp(i, _):
            pltpu.async_copy(zb, osh.at[pl.ds(tid * slc + i * zr, zr), :],
                             sem_s0)
            return 0
        lax.fori_loop(0, slc // zr, zcp, 0)
        pltpu.sync_copy(zb1, dsh.at[pl.ds(tid * slc, slc)])
        pltpu.sync_copy(att_hbm, att_v)

        def zwait(i, _):
            pltpu.make_async_copy(
                zb, osh.at[pl.ds(tid * slc + i * zr, zr), :], sem_s0).wait()
            return 0
        lax.fori_loop(0, slc // zr, zwait, 0)
        plsc.subcore_barrier()

        base = wid * epw
        brow = wid * nblk
        lanes = lax.iota(jnp.int32, L)
        attv = [att_v[pl.ds(q * L, L)] for q in range(D // L)]

        def fetch(b, idx, rows_s, rows_d, sem_s, sem_d):
            pltpu.sync_copy(sd_hbm.at[brow + b], idx)
            pltpu.async_copy(xl_hbm.at[idx.at[0]], rows_s, sem_s)
            pltpu.async_copy(xr_hbm.at[idx.at[1]], rows_d, sem_d)

        def drain(idx, rows_s, rows_d, sem_s, sem_d):
            pltpu.make_async_copy(xl_hbm.at[idx.at[0]], rows_s, sem_s).wait()
            pltpu.make_async_copy(xr_hbm.at[idx.at[1]], rows_d, sem_d).wait()

        def compute(b, idx, rows_s, rows_d, eev):
            off = pl.multiple_of(base + b * EB, EB)

            def sub(t, _):
                evec = jnp.zeros((L,), jnp.float32)
                for jj in range(L):
                    j = t * L + jj
                    vs = [rows_s[j, pl.ds(q * L, L)] for q in range(D // L)]
                    acc = jnp.zeros((L,), jnp.float32)
                    for q in range(D // L):
                        tt = vs[q] + rows_d[j, pl.ds(q * L, L)]
                        acc = acc + attv[q] * jnp.maximum(tt, 0.2 * tt)
                    ee = jnp.where(off + j < etot,
                                   jnp.exp(jnp.broadcast_to(jnp.sum(acc), (L,))),
                                   0.0)
                    evec = jnp.where(lanes == jj, ee, evec)
                    for q in range(D // L):
                        rows_s[j, pl.ds(q * L, L)] = vs[q] * ee
                eev[pl.ds(t * L, L)] = evec
                return 0
            lax.fori_loop(0, EB // L, sub, 0)

            pltpu.sync_copy(eev, dsh.at[idx.at[1]], add=True)
            pltpu.sync_copy(rows_s, osh.at[idx.at[1]], add=True)

        fetch(0, idx0, rows_s0, rows_d0, sem_s0, sem_d0)

        def blk2(ii, _):
            b0 = 2 * ii
            fetch(b0 + 1, idx1, rows_s1, rows_d1, sem_s1, sem_d1)
            drain(idx0, rows_s0, rows_d0, sem_s0, sem_d0)
            compute(b0, idx0, rows_s0, rows_d0, eev0)
            fetch(b0 + 2, idx0, rows_s0, rows_d0, sem_s0, sem_d0)
            drain(idx1, rows_s1, rows_d1, sem_s1, sem_d1)
            compute(b0 + 1, idx1, rows_s1, rows_d1, eev1)
            return 0
        lax.fori_loop(0, nblk // 2, blk2, 0)
        drain(idx0, rows_s0, rows_d0, sem_s0, sem_d0)

        plsc.subcore_barrier()
        pltpu.sync_copy(osh.at[pl.ds(tid * slc, slc), :],
                        out_hbm.at[c, pl.ds(tid * slc, slc), :])
        pltpu.sync_copy(dsh.at[pl.ds(tid * slc, slc)],
                        dparts_hbm.at[c, pl.ds(tid * slc, slc)])

    return fused


def kernel(x, edge_index, batch, Wl1, bl1, Wr1, br1, att1, b1,
           Wl2, bl2, Wr2, br2, att2, b2):
    n = x.shape[0]
    e = edge_index.shape[1]
    etot = e + n
    nblk_w = (etot + NW * EB - 1) // (NW * EB)
    nblk_w += nblk_w % 2
    epad = nblk_w * NW * EB
    np_ = ((n + NS * 128 - 1) // (NS * 128)) * (NS * 128)
    ng = 16

    loop = jnp.arange(n, dtype=jnp.int32)
    zpad = jnp.zeros((epad - etot + EB,), jnp.int32)
    s = jnp.concatenate([edge_index[0], loop, zpad])
    d = jnp.concatenate([edge_index[1], loop, zpad])
    sd = jnp.stack([s.reshape(-1, EB), d.reshape(-1, EB)], axis=1)

    sc_layer = _make_sc_layer(etot, epad, np_)
    mm2 = _make_mm2(n, 1000)
    merge_mm2 = _make_merge_mm2(n, 1000)
    merge_pool = _make_merge_pool(n, 1000, ng)

    b1r = b1.reshape(1, D)
    b2r = b2.reshape(1, D)

    xl1, xr1 = mm2(x, Wl1, bl1.reshape(1, D), Wr1, br1.reshape(1, D))
    oparts1, dparts1 = sc_layer(xl1, xr1, sd, att1.reshape(D))

    xl2, xr2 = merge_mm2(oparts1[0, :n], oparts1[1, :n],
                         dparts1[0, :n].reshape(n, 1),
                         dparts1[1, :n].reshape(n, 1), b1r,
                         Wl2, bl2.reshape(1, D), Wr2, br2.reshape(1, D))
    oparts2, dparts2 = sc_layer(xl2, xr2, sd, att2.reshape(D))

    pooled, h = merge_pool(oparts2[0, :n], oparts2[1, :n],
                           dparts2[0, :n].reshape(n, 1),
                           dparts2[1, :n].reshape(n, 1), b2r,
                           batch.reshape(n, 1))
    return (pooled, h)


# speedup vs baseline: 17.5640x; 1.0378x over previous
import functools

import jax
import jax.numpy as jnp
from jax import lax
from jax.experimental import pallas as pl
from jax.experimental.pallas import tpu as pltpu
from jax.experimental.pallas import tpu_sc as plsc

NC = 2
NS = 16
L = 16
NW = NC * NS
EB = 80
D = 128


def _mm2_body(x_ref, wl_ref, bl_ref, wr_ref, br_ref, xl_ref, xr_ref):
    x = x_ref[...]
    xl_ref[...] = jnp.dot(x, wl_ref[...], preferred_element_type=jnp.float32) + bl_ref[...]
    xr_ref[...] = jnp.dot(x, wr_ref[...], preferred_element_type=jnp.float32) + br_ref[...]


def _merge_mm2_body(pa_ref, pb_ref, da_ref, db_ref, b_ref, wl_ref, bl_ref,
                    wr_ref, br_ref, xl_ref, xr_ref):
    dn = da_ref[0] + db_ref[0] + 1e-16
    h = jnp.maximum((pa_ref[0] + pb_ref[0]) / dn + b_ref[...], 0.0)
    xl_ref[...] = jnp.dot(h, wl_ref[...], preferred_element_type=jnp.float32) + bl_ref[...]
    xr_ref[...] = jnp.dot(h, wr_ref[...], preferred_element_type=jnp.float32) + br_ref[...]


def _make_mm2(n, rb):
    grid = n // rb
    full = lambda i: (0, 0)
    row = lambda i: (i, 0)
    return pl.pallas_call(
        _mm2_body,
        grid=(grid,),
        in_specs=[
            pl.BlockSpec((rb, D), row),
            pl.BlockSpec((D, D), full),
            pl.BlockSpec((1, D), full),
            pl.BlockSpec((D, D), full),
            pl.BlockSpec((1, D), full),
        ],
        out_specs=[pl.BlockSpec((rb, D), row), pl.BlockSpec((rb, D), row)],
        out_shape=[
            jax.ShapeDtypeStruct((n, D), jnp.float32),
            jax.ShapeDtypeStruct((n, D), jnp.float32),
        ],
    )


def _make_merge_mm2(n, rb):
    grid = n // rb
    full = lambda i: (0, 0)
    row = lambda i: (i, 0)
    return pl.pallas_call(
        _merge_mm2_body,
        grid=(grid,),
        in_specs=[
            pl.BlockSpec((1, rb, D), lambda i: (0, i, 0)),
            pl.BlockSpec((1, rb, D), lambda i: (1, i, 0)),
            pl.BlockSpec((1, rb, 1), lambda i: (0, i, 0)),
            pl.BlockSpec((1, rb, 1), lambda i: (1, i, 0)),
            pl.BlockSpec((1, D), full),
            pl.BlockSpec((D, D), full),
            pl.BlockSpec((1, D), full),
            pl.BlockSpec((D, D), full),
            pl.BlockSpec((1, D), full),
        ],
        out_specs=[pl.BlockSpec((rb, D), row), pl.BlockSpec((rb, D), row)],
        out_shape=[
            jax.ShapeDtypeStruct((n, D), jnp.float32),
            jax.ShapeDtypeStruct((n, D), jnp.float32),
        ],
    )


def _make_merge_pool(n, rb, ng):
    grid = n // rb

    def body(pa_ref, pb_ref, da_ref, db_ref, b_ref, bt_ref, pooled_ref, h_ref,
             acc, cnt):
        i = pl.program_id(0)
        dnm = da_ref[0] + db_ref[0] + 1e-16
        h = (pa_ref[0] + pb_ref[0]) / dnm + b_ref[...]
        h_ref[...] = h
        groups = lax.broadcasted_iota(jnp.int32, (rb, ng), 1)
        oh = (bt_ref[...] == groups).astype(jnp.float32)
        dn = (((0,), (0,)), ((), ()))
        ps = lax.dot_general(oh, h, dn, preferred_element_type=jnp.float32)
        cs = lax.dot_general(oh, jnp.ones((rb, D), jnp.float32), dn,
                             preferred_element_type=jnp.float32)

        @pl.when(i == 0)
        def _():
            acc[...] = ps
            cnt[...] = cs

        @pl.when(i > 0)
        def _():
            acc[...] = acc[...] + ps
            cnt[...] = cnt[...] + cs

        @pl.when(i == grid - 1)
        def _():
            pooled_ref[...] = acc[...] / jnp.maximum(cnt[...], 1.0)

    full = lambda i: (0, 0)
    row = lambda i: (i, 0)
    return pl.pallas_call(
        body,
        grid=(grid,),
        in_specs=[
            pl.BlockSpec((1, rb, D), lambda i: (0, i, 0)),
            pl.BlockSpec((1, rb, D), lambda i: (1, i, 0)),
            pl.BlockSpec((1, rb, 1), lambda i: (0, i, 0)),
            pl.BlockSpec((1, rb, 1), lambda i: (1, i, 0)),
            pl.BlockSpec((1, D), full),
            pl.BlockSpec((rb, 1), row),
        ],
        out_specs=[pl.BlockSpec((ng, D), full), pl.BlockSpec((rb, D), row)],
        out_shape=[
            jax.ShapeDtypeStruct((ng, D), jnp.float32),
            jax.ShapeDtypeStruct((n, D), jnp.float32),
        ],
        scratch_shapes=[
            pltpu.VMEM((ng, D), jnp.float32),
            pltpu.VMEM((ng, D), jnp.float32),
        ],
    )


def _leaky(t):
    return jnp.maximum(t, 0.0) + 0.2 * jnp.minimum(t, 0.0)


@functools.lru_cache(maxsize=None)
def _make_sc_layer(etot, epad, np_):
    nblk = epad // (NW * EB)
    assert nblk % 2 == 0
    epw = nblk * EB
    slc = np_ // NS
    zr = 16
    mesh = plsc.VectorSubcoreMesh(core_axis_name="c", subcore_axis_name="s",
                                  num_cores=NC, num_subcores=NS)

    @functools.partial(
        pl.kernel,
        out_type=[
            jax.ShapeDtypeStruct((NC, np_, D), jnp.float32),
            jax.ShapeDtypeStruct((NC, np_), jnp.float32),
        ],
        mesh=mesh,
        compiler_params=pltpu.CompilerParams(needs_layout_passes=False),
        scratch_types=[
            pltpu.VMEM((D,), jnp.float32),
            pltpu.VMEM((2, EB), jnp.int32),
            pltpu.VMEM((EB, D), jnp.float32),
            pltpu.VMEM((EB, D), jnp.float32),
            pltpu.VMEM((EB,), jnp.float32),
            pltpu.VMEM((2, EB), jnp.int32),
            pltpu.VMEM((EB, D), jnp.float32),
            pltpu.VMEM((EB, D), jnp.float32),
            pltpu.VMEM((EB,), jnp.float32),
            pltpu.VMEM((zr, D), jnp.float32),
            pltpu.VMEM((slc,), jnp.float32),
            pltpu.VMEM_SHARED((np_, D), jnp.float32),
            pltpu.VMEM_SHARED((np_,), jnp.float32),
            pltpu.SemaphoreType.DMA,
            pltpu.SemaphoreType.DMA,
            pltpu.SemaphoreType.DMA,
            pltpu.SemaphoreType.DMA,
        ],
    )
    def fused(xl_hbm, xr_hbm, sd_hbm, att_hbm, out_hbm, dparts_hbm,
              att_v, idx0, rows_s0, rows_d0, eev0,
              idx1, rows_s1, rows_d1, eev1, zb, zb1, osh, dsh,
              sem_s0, sem_d0, sem_s1, sem_d1):
        c = lax.axis_index("c")
        tid = lax.axis_index("s")
        wid = tid * NC + c

        def zrow(i, _):
            for q in range(D // L):
                zb[i, pl.ds(q * L, L)] = jnp.zeros((L,), jnp.float32)
            return 0
        lax.fori_loop(0, zr, zrow, 0)

        def z1(i, _):
            zb1[pl.ds(i * L, L)] = jnp.zeros((L,), jnp.float32)
            return 0
        lax.fori_loop(0, slc // L, z1, 0)

        def zcp(i, _):
            pltpu.async_copy(zb, osh.at[pl.ds(tid * slc + i * zr, zr), :],
                             sem_s0)
            return 0
        lax.fori_loop(0, slc // zr, zcp, 0)
        pltpu.sync_copy(zb1, dsh.at[pl.ds(tid * slc, slc)])
        pltpu.sync_copy(att_hbm, att_v)

        def zwait(i, _):
            pltpu.make_async_copy(
                zb, osh.at[pl.ds(tid * slc + i * zr, zr), :], sem_s0).wait()
            return 0
        lax.fori_loop(0, slc // zr, zwait, 0)
        plsc.subcore_barrier()

        base = wid * epw
        brow = wid * nblk
        lanes = lax.iota(jnp.int32, L)
        attv = [att_v[pl.ds(q * L, L)] for q in range(D // L)]

        def fetch(b, idx, rows_s, rows_d, sem_s, sem_d):
            pltpu.sync_copy(sd_hbm.at[brow + b], idx)
            pltpu.async_copy(xl_hbm.at[idx.at[0]], rows_s, sem_s)
            pltpu.async_copy(xr_hbm.at[idx.at[1]], rows_d, sem_d)

        def drain(idx, rows_s, rows_d, sem_s, sem_d):
            pltpu.make_async_copy(xl_hbm.at[idx.at[0]], rows_s, sem_s).wait()
            pltpu.make_async_copy(xr_hbm.at[idx.at[1]], rows_d, sem_d).wait()

        def compute(b, idx, rows_s, rows_d, eev):
            off = pl.multiple_of(base + b * EB, EB)

            def sub(t, _):
                evec = jnp.zeros((L,), jnp.float32)
                for jj in range(L):
                    j = t * L + jj
                    vs = [rows_s[j, pl.ds(q * L, L)] for q in range(D // L)]
                    acc = jnp.zeros((L,), jnp.float32)
                    for q in range(D // L):
                        tt = vs[q] + rows_d[j, pl.ds(q * L, L)]
                        acc = acc + attv[q] * jnp.maximum(tt, 0.2 * tt)
                    ee = jnp.where(off + j < etot,
                                   jnp.exp(jnp.broadcast_to(jnp.sum(acc), (L,))),
                                   0.0)
                    evec = jnp.where(lanes == jj, ee, evec)
                    for q in range(D // L):
                        rows_s[j, pl.ds(q * L, L)] = vs[q] * ee
                eev[pl.ds(t * L, L)] = evec
                return 0
            lax.fori_loop(0, EB // L, sub, 0)

            pltpu.sync_copy(eev, dsh.at[idx.at[1]], add=True)
            pltpu.sync_copy(rows_s, osh.at[idx.at[1]], add=True)

        fetch(0, idx0, rows_s0, rows_d0, sem_s0, sem_d0)

        def blk2(ii, _):
            b0 = 2 * ii
            fetch(b0 + 1, idx1, rows_s1, rows_d1, sem_s1, sem_d1)
            drain(idx0, rows_s0, rows_d0, sem_s0, sem_d0)
            compute(b0, idx0, rows_s0, rows_d0, eev0)
            fetch(b0 + 2, idx0, rows_s0, rows_d0, sem_s0, sem_d0)
            drain(idx1, rows_s1, rows_d1, sem_s1, sem_d1)
            compute(b0 + 1, idx1, rows_s1, rows_d1, eev1)
            return 0
        lax.fori_loop(0, nblk // 2, blk2, 0)
        drain(idx0, rows_s0, rows_d0, sem_s0, sem_d0)

        plsc.subcore_barrier()
        pltpu.sync_copy(osh.at[pl.ds(tid * slc, slc), :],
                        out_hbm.at[c, pl.ds(tid * slc, slc), :])
        pltpu.sync_copy(dsh.at[pl.ds(tid * slc, slc)],
                        dparts_hbm.at[c, pl.ds(tid * slc, slc)])

    return fused


def kernel(x, edge_index, batch, Wl1, bl1, Wr1, br1, att1, b1,
           Wl2, bl2, Wr2, br2, att2, b2):
    n = x.shape[0]
    e = edge_index.shape[1]
    etot = e + n
    nblk_w = (etot + NW * EB - 1) // (NW * EB)
    nblk_w += nblk_w % 2
    epad = nblk_w * NW * EB
    np_ = ((n + NS * 128 - 1) // (NS * 128)) * (NS * 128)
    ng = 16

    loop = jnp.arange(n, dtype=jnp.int32)
    zpad = jnp.zeros((epad - etot + EB,), jnp.int32)
    s = jnp.concatenate([edge_index[0], loop, zpad])
    d = jnp.concatenate([edge_index[1], loop, zpad])
    sd = jnp.stack([s.reshape(-1, EB), d.reshape(-1, EB)], axis=1)

    sc_layer = _make_sc_layer(etot, epad, np_)
    mm2 = _make_mm2(n, 1000)
    merge_mm2 = _make_merge_mm2(n, 1000)
    merge_pool = _make_merge_pool(n, 1000, ng)

    b1r = b1.reshape(1, D)
    b2r = b2.reshape(1, D)

    xl1, xr1 = mm2(x, Wl1, bl1.reshape(1, D), Wr1, br1.reshape(1, D))
    oparts1, dparts1 = sc_layer(xl1, xr1, sd, att1.reshape(D))

    dparts1r = dparts1.reshape(NC, -1, 1)
    xl2, xr2 = merge_mm2(oparts1, oparts1, dparts1r, dparts1r, b1r,
                         Wl2, bl2.reshape(1, D), Wr2, br2.reshape(1, D))
    oparts2, dparts2 = sc_layer(xl2, xr2, sd, att2.reshape(D))

    dparts2r = dparts2.reshape(NC, -1, 1)
    pooled, h = merge_pool(oparts2, oparts2, dparts2r, dparts2r, b2r,
                           batch.reshape(n, 1))
    return (pooled, h)
